# Initial kernel scaffold; baseline (speedup 1.0000x reference)
import functools

import jax
import jax.numpy as jnp
from jax import lax
from jax.experimental import pallas as pl
from jax.experimental.pallas import tpu as pltpu
from jax.experimental.pallas import tpu_sc as plsc

N = 10000
E = 320000
C = 128
ND = 3
FC = 128

NC, NS = 2, 16
NW = NC * NS
EW = E // NW
CH = 80
NCHUNK = EW // CH

NP = 10240
ROWS_PER_TILE = NP // NS

T1W = 144
T2W = 16

EB = 2000
GRID = E // EB

_mesh = plsc.VectorSubcoreMesh(core_axis_name="c", subcore_axis_name="s")


@functools.partial(
    pl.kernel,
    out_type=(
        jax.ShapeDtypeStruct((E, T1W), jnp.float32),
        jax.ShapeDtypeStruct((E, T2W), jnp.float32),
    ),
    mesh=_mesh,
    scratch_types=(
        pltpu.VMEM((NCHUNK, CH), jnp.int32),
        pltpu.VMEM((NCHUNK, CH), jnp.int32),
        pltpu.VMEM((CH, T1W), jnp.float32),
        pltpu.VMEM((CH, T2W), jnp.float32),
        pltpu.SemaphoreType.DMA,
        pltpu.SemaphoreType.DMA,
    ),
)
def _gather_stage(t1_hbm, t2_hbm, src_hbm, tgt_hbm, g1_hbm, g2_hbm,
                  src_v, tgt_v, buf1, buf2, sem1, sem2):
    wid = lax.axis_index("c") * NS + lax.axis_index("s")
    base = wid * EW
    pltpu.sync_copy(src_hbm.at[wid], src_v)
    pltpu.sync_copy(tgt_hbm.at[wid], tgt_v)

    @pl.loop(0, NCHUNK)
    def _chunks(j):
        cp1 = pltpu.async_copy(t1_hbm.at[src_v.at[j]], buf1, sem1)
        cp2 = pltpu.async_copy(t2_hbm.at[tgt_v.at[j]], buf2, sem2)
        cp1.wait()
        cp2.wait()
        eb = base + j * CH
        pltpu.sync_copy(buf1, g1_hbm.at[pl.ds(eb, CH)])
        pltpu.sync_copy(buf2, g2_hbm.at[pl.ds(eb, CH)])


def _mlp_body(g1_ref, g2_ref, w_ref, w1_ref, b1_ref, w2_ref, b2_ref, out_ref):
    g1 = g1_ref[...]
    g2 = g2_ref[...]
    h = jnp.broadcast_to(b1_ref[...][None, :], (EB, FC))
    for d in range(ND):
        h = h + g1[:, C + d:C + d + 1] * w1_ref[d:d + 1, :]
    for d in range(ND):
        h = h + g2[:, d:d + 1] * w1_ref[ND + d:ND + d + 1, :]
    h = 0.5 * h * (1.0 + lax.erf(h * 0.7071067811865476))
    ker = jnp.dot(h, w2_ref[...], preferred_element_type=jnp.float32)
    ker = ker + b2_ref[...][None, :]
    out_ref[...] = ker * (w_ref[...] * g1[:, :C])


def _mlp_stage(g1, g2, w, W1, b1, W2, b2):
    return pl.pallas_call(
        _mlp_body,
        grid=(GRID,),
        in_specs=[
            pl.BlockSpec((EB, T1W), lambda i: (i, 0)),
            pl.BlockSpec((EB, T2W), lambda i: (i, 0)),
            pl.BlockSpec((EB, 1), lambda i: (i, 0)),
            pl.BlockSpec((2 * ND, FC), lambda i: (0, 0)),
            pl.BlockSpec((FC,), lambda i: (0,)),
            pl.BlockSpec((FC, C), lambda i: (0, 0)),
            pl.BlockSpec((C,), lambda i: (0,)),
        ],
        out_specs=pl.BlockSpec((EB, C), lambda i: (i, 0)),
        out_shape=jax.ShapeDtypeStruct((E, C), jnp.float32),
        compiler_params=pltpu.CompilerParams(
            dimension_semantics=("arbitrary",),
        ),
    )(g1, g2, w, W1, b1, W2, b2)


@functools.partial(
    pl.kernel,
    out_type=jax.ShapeDtypeStruct((NC, NP, C), jnp.float32),
    mesh=_mesh,
    scratch_types=(
        pltpu.VMEM((NCHUNK, CH), jnp.int32),
        pltpu.VMEM((CH, C), jnp.float32),
        pltpu.VMEM((16, C), jnp.float32),
        pltpu.VMEM_SHARED((NP, C), jnp.float32),
    ),
)
def _scatter_stage(feat_hbm, tgt_hbm, out_hbm, tgt_v, fbuf, zbuf, acc):
    cid = lax.axis_index("c")
    sid = lax.axis_index("s")
    wid = cid * NS + sid

    for r in range(16):
        for cc in range(C // 16):
            zbuf[r, pl.ds(cc * 16, 16)] = jnp.zeros((16,), jnp.float32)
    row0 = sid * ROWS_PER_TILE

    @pl.loop(0, ROWS_PER_TILE // 16)
    def _zero(j):
        pltpu.sync_copy(zbuf, acc.at[pl.ds(row0 + j * 16, 16)])

    plsc.subcore_barrier()

    pltpu.sync_copy(tgt_hbm.at[wid], tgt_v)
    base = wid * EW

    @pl.loop(0, NCHUNK)
    def _chunks(j):
        pltpu.sync_copy(feat_hbm.at[pl.ds(base + j * CH, CH)], fbuf)
        pltpu.sync_copy(fbuf, acc.at[tgt_v.at[j]], add=True)

    plsc.subcore_barrier()
    pltpu.sync_copy(acc.at[pl.ds(row0, ROWS_PER_TILE)],
                    out_hbm.at[cid, pl.ds(row0, ROWS_PER_TILE)])


def kernel(nodes_x, nodes_y, x, directed_edges, weights, W1, b1, W2, b2):
    xT = jnp.transpose(x[0], (1, 0))
    padx = jnp.zeros((N, T1W - C - ND), jnp.float32)
    t1 = jnp.concatenate([xT, nodes_x[0], padx], axis=1)
    pady = jnp.zeros((N, T2W - ND), jnp.float32)
    t2 = jnp.concatenate([nodes_y[0], pady], axis=1)

    src = directed_edges[0, :, 0].reshape(NW, NCHUNK, CH)
    tgt = directed_edges[0, :, 1].reshape(NW, NCHUNK, CH)

    g1, g2 = _gather_stage(t1, t2, src, tgt)
    feat = _mlp_stage(g1, g2, weights[0].reshape(E, 1), W1, b1, W2, b2)
    partial = _scatter_stage(feat, tgt)
    out = partial[0, :N] + partial[1, :N]
    return jnp.transpose(out, (1, 0))[None]


# speedup vs baseline: 15.2588x; 15.2588x over previous
import functools

import jax
import jax.numpy as jnp
from jax import lax
from jax.experimental import pallas as pl
from jax.experimental.pallas import tpu as pltpu
from jax.experimental.pallas import tpu_sc as plsc

N = 10000
E = 320000
C = 128
ND = 3
FC = 128

NC, NS = 2, 16
NW = NC * NS
EW = E // NW
CH = 80
NCHUNK = EW // CH

NP = 10240
ROWS_PER_TILE = NP // NS

CW = 16

EB = 2000
GRID = E // EB

_mesh = plsc.VectorSubcoreMesh(core_axis_name="c", subcore_axis_name="s")


@functools.partial(
    pl.kernel,
    out_type=jax.ShapeDtypeStruct((E, C), jnp.float32),
    mesh=_mesh,
    scratch_types=(
        pltpu.VMEM((NCHUNK, CH), jnp.int32),
        pltpu.VMEM((CH, C), jnp.float32),
        pltpu.SemaphoreType.DMA,
    ),
)
def _gather_x(xt_hbm, src_hbm, gx_hbm, src_v, buf, sem):
    wid = lax.axis_index("c") * NS + lax.axis_index("s")
    base = wid * EW
    pltpu.sync_copy(src_hbm.at[wid], src_v)

    @pl.loop(0, NCHUNK)
    def _chunks(j):
        pltpu.async_copy(xt_hbm.at[src_v.at[j]], buf, sem).wait()
        pltpu.sync_copy(buf, gx_hbm.at[pl.ds(base + j * CH, CH)])


@functools.partial(
    pl.kernel,
    out_type=(
        jax.ShapeDtypeStruct((E, CW), jnp.float32),
        jax.ShapeDtypeStruct((E, CW), jnp.float32),
    ),
    mesh=_mesh,
    scratch_types=(
        pltpu.VMEM((NCHUNK, CH), jnp.int32),
        pltpu.VMEM((NCHUNK, CH), jnp.int32),
        pltpu.VMEM((CH, CW), jnp.float32),
        pltpu.VMEM((CH, CW), jnp.float32),
        pltpu.SemaphoreType.DMA,
        pltpu.SemaphoreType.DMA,
    ),
    compiler_params=pltpu.CompilerParams(use_tc_tiling_on_sc=False),
)
def _gather_coords(t1_hbm, t2_hbm, src_hbm, tgt_hbm, g1_hbm, g2_hbm,
                   src_v, tgt_v, buf1, buf2, sem1, sem2):
    wid = lax.axis_index("c") * NS + lax.axis_index("s")
    base = wid * EW
    pltpu.sync_copy(src_hbm.at[wid], src_v)
    pltpu.sync_copy(tgt_hbm.at[wid], tgt_v)

    @pl.loop(0, NCHUNK)
    def _chunks(j):
        cp1 = pltpu.async_copy(t1_hbm.at[src_v.at[j]], buf1, sem1)
        cp2 = pltpu.async_copy(t2_hbm.at[tgt_v.at[j]], buf2, sem2)
        cp1.wait()
        cp2.wait()
        eb = base + j * CH
        pltpu.sync_copy(buf1, g1_hbm.at[pl.ds(eb, CH)])
        pltpu.sync_copy(buf2, g2_hbm.at[pl.ds(eb, CH)])


def _mlp_body(g1_ref, g2_ref, gx_ref, w_ref, w1a_ref, w1b_ref, b1_ref,
              w2_ref, b2_ref, out_ref):
    g1 = g1_ref[...]
    g2 = g2_ref[...]
    h = jnp.dot(g1, w1a_ref[...], preferred_element_type=jnp.float32)
    h = h + jnp.dot(g2, w1b_ref[...], preferred_element_type=jnp.float32)
    h = h + b1_ref[...][None, :]
    h = 0.5 * h * (1.0 + lax.erf(h * 0.7071067811865476))
    ker = jnp.dot(h, w2_ref[...], preferred_element_type=jnp.float32)
    ker = ker + b2_ref[...][None, :]
    out_ref[...] = ker * (w_ref[...] * gx_ref[...])


def _mlp_stage(g1, g2, gx, w, w1a, w1b, b1, W2, b2):
    return pl.pallas_call(
        _mlp_body,
        grid=(GRID,),
        in_specs=[
            pl.BlockSpec((EB, CW), lambda i: (i, 0)),
            pl.BlockSpec((EB, CW), lambda i: (i, 0)),
            pl.BlockSpec((EB, C), lambda i: (i, 0)),
            pl.BlockSpec((EB, 1), lambda i: (i, 0)),
            pl.BlockSpec((CW, FC), lambda i: (0, 0)),
            pl.BlockSpec((CW, FC), lambda i: (0, 0)),
            pl.BlockSpec((FC,), lambda i: (0,)),
            pl.BlockSpec((FC, C), lambda i: (0, 0)),
            pl.BlockSpec((C,), lambda i: (0,)),
        ],
        out_specs=pl.BlockSpec((EB, C), lambda i: (i, 0)),
        out_shape=jax.ShapeDtypeStruct((E, C), jnp.float32),
        compiler_params=pltpu.CompilerParams(
            dimension_semantics=("arbitrary",),
        ),
    )(g1, g2, gx, w, w1a, w1b, b1, W2, b2)


@functools.partial(
    pl.kernel,
    out_type=jax.ShapeDtypeStruct((NC, NP, C), jnp.float32),
    mesh=_mesh,
    scratch_types=(
        pltpu.VMEM((NCHUNK, CH), jnp.int32),
        pltpu.VMEM((CH, C), jnp.float32),
        pltpu.VMEM((16, C), jnp.float32),
        pltpu.VMEM_SHARED((NP, C), jnp.float32),
    ),
)
def _scatter_stage(feat_hbm, tgt_hbm, out_hbm, tgt_v, fbuf, zbuf, acc):
    cid = lax.axis_index("c")
    sid = lax.axis_index("s")
    wid = cid * NS + sid

    for r in range(16):
        for cc in range(C // 16):
            zbuf[r, pl.ds(cc * 16, 16)] = jnp.zeros((16,), jnp.float32)
    row0 = sid * ROWS_PER_TILE

    @pl.loop(0, ROWS_PER_TILE // 16)
    def _zero(j):
        pltpu.sync_copy(zbuf, acc.at[pl.ds(row0 + j * 16, 16)])

    plsc.subcore_barrier()

    pltpu.sync_copy(tgt_hbm.at[wid], tgt_v)
    base = wid * EW

    @pl.loop(0, NCHUNK)
    def _chunks(j):
        pltpu.sync_copy(feat_hbm.at[pl.ds(base + j * CH, CH)], fbuf)
        pltpu.sync_copy(fbuf, acc.at[tgt_v.at[j]], add=True)

    plsc.subcore_barrier()
    pltpu.sync_copy(acc.at[pl.ds(row0, ROWS_PER_TILE)],
                    out_hbm.at[cid, pl.ds(row0, ROWS_PER_TILE)])


def kernel(nodes_x, nodes_y, x, directed_edges, weights, W1, b1, W2, b2):
    xT = jnp.transpose(x[0], (1, 0))
    pad = jnp.zeros((N, CW - ND), jnp.float32)
    t1 = jnp.concatenate([nodes_x[0], pad], axis=1)
    t2 = jnp.concatenate([nodes_y[0], pad], axis=1)

    src = directed_edges[0, :, 0].reshape(NW, NCHUNK, CH)
    tgt = directed_edges[0, :, 1].reshape(NW, NCHUNK, CH)

    wpad = jnp.zeros((CW - ND, FC), jnp.float32)
    w1a = jnp.concatenate([W1[:ND], wpad], axis=0)
    w1b = jnp.concatenate([W1[ND:], wpad], axis=0)

    gx = _gather_x(xT, src)
    g1, g2 = _gather_coords(t1, t2, src, tgt)
    feat = _mlp_stage(g1, g2, gx, weights[0].reshape(E, 1), w1a, w1b, b1, W2, b2)
    partial = _scatter_stage(feat, tgt)
    out = partial[0, :N] + partial[1, :N]
    return jnp.transpose(out, (1, 0))[None]


# speedup vs baseline: 17.8534x; 1.1700x over previous
import functools

import jax
import jax.numpy as jnp
from jax import lax
from jax.experimental import pallas as pl
from jax.experimental.pallas import tpu as pltpu
from jax.experimental.pallas import tpu_sc as plsc

N = 10000
E = 320000
C = 128
ND = 3
FC = 128

NC, NS = 2, 16
NW = NC * NS
EW = E // NW
CH = 80
NCHUNK = EW // CH

NP = 10240
ROWS_PER_TILE = NP // NS

CW = 16

EB = 2000
GRID = E // EB

_mesh = plsc.VectorSubcoreMesh(core_axis_name="c", subcore_axis_name="s")


NBUF = 4


@functools.partial(
    pl.kernel,
    out_type=jax.ShapeDtypeStruct((E, C), jnp.float32),
    mesh=_mesh,
    scratch_types=(
        (pltpu.VMEM((NCHUNK, CH), jnp.int32),)
        + (pltpu.VMEM((CH, C), jnp.float32),) * NBUF
        + (pltpu.SemaphoreType.DMA,) * (2 * NBUF)
    ),
)
def _gather_x(xt_hbm, src_hbm, gx_hbm, src_v, *bs):
    bufs, gsems, wsems = bs[:NBUF], bs[NBUF:2 * NBUF], bs[2 * NBUF:]
    wid = lax.axis_index("c") * NS + lax.axis_index("s")
    base = wid * EW
    pltpu.sync_copy(src_hbm.at[wid], src_v)

    def g_cp(j, b):
        return pltpu.make_async_copy(xt_hbm.at[src_v.at[j]], bufs[b], gsems[b])

    def w_cp(j, b):
        return pltpu.make_async_copy(
            bufs[b], gx_hbm.at[pl.ds(base + j * CH, CH)], wsems[b])

    for b in range(NBUF - 1):
        g_cp(b, b).start()

    @pl.loop(0, NCHUNK + NBUF - 1 - ((NCHUNK - 1) % NBUF), step=NBUF)
    def _chunks(j):
        for b in range(NBUF):
            jj = j + b
            nxt = jj + NBUF - 1

            @pl.when(jj < NCHUNK)
            def _():
                g_cp(jj, b).wait()
                w_cp(jj, b).start()

            @pl.when(nxt < NCHUNK)
            def _():
                bn = (b + NBUF - 1) % NBUF

                @pl.when(nxt >= NBUF)
                def _():
                    w_cp(nxt - NBUF, bn).wait()

                g_cp(nxt, bn).start()

    for b in range(NBUF):
        jlast = NCHUNK - NBUF + ((b - NCHUNK) % NBUF)
        w_cp(jlast, b).wait()


@functools.partial(
    pl.kernel,
    out_type=(
        jax.ShapeDtypeStruct((E, CW), jnp.float32),
        jax.ShapeDtypeStruct((E, CW), jnp.float32),
    ),
    mesh=_mesh,
    scratch_types=(
        (pltpu.VMEM((NCHUNK, CH), jnp.int32),) * 2
        + (pltpu.VMEM((CH, CW), jnp.float32),) * (2 * NBUF)
        + (pltpu.SemaphoreType.DMA,) * (4 * NBUF)
    ),
    compiler_params=pltpu.CompilerParams(use_tc_tiling_on_sc=False),
)
def _gather_coords(t1_hbm, t2_hbm, src_hbm, tgt_hbm, g1_hbm, g2_hbm,
                   src_v, tgt_v, *bs):
    buf1, buf2 = bs[:NBUF], bs[NBUF:2 * NBUF]
    gs1 = bs[2 * NBUF:3 * NBUF]
    gs2 = bs[3 * NBUF:4 * NBUF]
    ws1 = bs[4 * NBUF:5 * NBUF]
    ws2 = bs[5 * NBUF:6 * NBUF]
    wid = lax.axis_index("c") * NS + lax.axis_index("s")
    base = wid * EW
    pltpu.sync_copy(src_hbm.at[wid], src_v)
    pltpu.sync_copy(tgt_hbm.at[wid], tgt_v)

    def g_cp(j, b):
        return (pltpu.make_async_copy(t1_hbm.at[src_v.at[j]], buf1[b], gs1[b]),
                pltpu.make_async_copy(t2_hbm.at[tgt_v.at[j]], buf2[b], gs2[b]))

    def w_cp(j, b):
        eb = base + j * CH
        return (pltpu.make_async_copy(buf1[b], g1_hbm.at[pl.ds(eb, CH)], ws1[b]),
                pltpu.make_async_copy(buf2[b], g2_hbm.at[pl.ds(eb, CH)], ws2[b]))

    for b in range(NBUF - 1):
        for cp in g_cp(b, b):
            cp.start()

    @pl.loop(0, NCHUNK + NBUF - 1 - ((NCHUNK - 1) % NBUF), step=NBUF)
    def _chunks(j):
        for b in range(NBUF):
            jj = j + b
            nxt = jj + NBUF - 1

            @pl.when(jj < NCHUNK)
            def _():
                for cp in g_cp(jj, b):
                    cp.wait()
                for cp in w_cp(jj, b):
                    cp.start()

            @pl.when(nxt < NCHUNK)
            def _():
                bn = (b + NBUF - 1) % NBUF

                @pl.when(nxt >= NBUF)
                def _():
                    for cp in w_cp(nxt - NBUF, bn):
                        cp.wait()

                for cp in g_cp(nxt, bn):
                    cp.start()

    for b in range(NBUF):
        jlast = NCHUNK - NBUF + ((b - NCHUNK) % NBUF)
        for cp in w_cp(jlast, b):
            cp.wait()


def _mlp_body(g1_ref, g2_ref, gx_ref, w_ref, w1a_ref, w1b_ref, b1_ref,
              w2_ref, b2_ref, out_ref):
    g1 = g1_ref[...]
    g2 = g2_ref[...]
    h = jnp.dot(g1, w1a_ref[...], preferred_element_type=jnp.float32)
    h = h + jnp.dot(g2, w1b_ref[...], preferred_element_type=jnp.float32)
    h = h + b1_ref[...][None, :]
    h = 0.5 * h * (1.0 + lax.erf(h * 0.7071067811865476))
    ker = jnp.dot(h, w2_ref[...], preferred_element_type=jnp.float32)
    ker = ker + b2_ref[...][None, :]
    out_ref[...] = ker * (w_ref[...] * gx_ref[...])


def _mlp_stage(g1, g2, gx, w, w1a, w1b, b1, W2, b2):
    return pl.pallas_call(
        _mlp_body,
        grid=(GRID,),
        in_specs=[
            pl.BlockSpec((EB, CW), lambda i: (i, 0)),
            pl.BlockSpec((EB, CW), lambda i: (i, 0)),
            pl.BlockSpec((EB, C), lambda i: (i, 0)),
            pl.BlockSpec((EB, 1), lambda i: (i, 0)),
            pl.BlockSpec((CW, FC), lambda i: (0, 0)),
            pl.BlockSpec((CW, FC), lambda i: (0, 0)),
            pl.BlockSpec((FC,), lambda i: (0,)),
            pl.BlockSpec((FC, C), lambda i: (0, 0)),
            pl.BlockSpec((C,), lambda i: (0,)),
        ],
        out_specs=pl.BlockSpec((EB, C), lambda i: (i, 0)),
        out_shape=jax.ShapeDtypeStruct((E, C), jnp.float32),
        compiler_params=pltpu.CompilerParams(
            dimension_semantics=("arbitrary",),
        ),
    )(g1, g2, gx, w, w1a, w1b, b1, W2, b2)


NBUF_S = 3

@functools.partial(
    pl.kernel,
    out_type=jax.ShapeDtypeStruct((NC, NP, C), jnp.float32),
    mesh=_mesh,
    scratch_types=(
        (pltpu.VMEM((NCHUNK, CH), jnp.int32),
         pltpu.VMEM((16, C), jnp.float32),
         pltpu.VMEM_SHARED((NP, C), jnp.float32),
         pltpu.SemaphoreType.DMA)
        + (pltpu.VMEM((CH, C), jnp.float32),) * NBUF_S
        + (pltpu.SemaphoreType.DMA,) * (2 * NBUF_S)
    ),
)
def _scatter_stage(feat_hbm, tgt_hbm, out_hbm, tgt_v, zbuf, acc, zsem, *bs):
    fbuf, rsems, ssems = bs[:NBUF_S], bs[NBUF_S:2 * NBUF_S], bs[2 * NBUF_S:]
    cid = lax.axis_index("c")
    sid = lax.axis_index("s")
    wid = cid * NS + sid

    for r in range(16):
        for cc in range(C // 16):
            zbuf[r, pl.ds(cc * 16, 16)] = jnp.zeros((16,), jnp.float32)
    row0 = sid * ROWS_PER_TILE

    @pl.loop(0, ROWS_PER_TILE // 16)
    def _zero(j):
        pltpu.async_copy(zbuf, acc.at[pl.ds(row0 + j * 16, 16)], zsem)

    @pl.loop(0, ROWS_PER_TILE // 16)
    def _zdrain(j):
        pltpu.make_async_copy(zbuf, acc.at[pl.ds(row0 + j * 16, 16)],
                              zsem).wait()

    plsc.subcore_barrier()

    pltpu.sync_copy(tgt_hbm.at[wid], tgt_v)
    base = wid * EW

    def r_cp(j, b):
        return pltpu.make_async_copy(
            feat_hbm.at[pl.ds(base + j * CH, CH)], fbuf[b], rsems[b])

    def s_cp(j, b):
        return pltpu.make_async_copy(fbuf[b], acc.at[tgt_v.at[j]], ssems[b])

    for b in range(NBUF_S - 1):
        r_cp(b, b).start()

    @pl.loop(0, NCHUNK + NBUF_S - 1 - ((NCHUNK - 1) % NBUF_S), step=NBUF_S)
    def _chunks(j):
        for b in range(NBUF_S):
            jj = j + b
            nxt = jj + NBUF_S - 1

            @pl.when(jj < NCHUNK)
            def _():
                r_cp(jj, b).wait()
                pltpu.async_copy(fbuf[b], acc.at[tgt_v.at[jj]], ssems[b],
                                 add=True)

            @pl.when(nxt < NCHUNK)
            def _():
                bn = (b + NBUF_S - 1) % NBUF_S

                @pl.when(nxt >= NBUF_S)
                def _():
                    s_cp(nxt - NBUF_S, bn).wait()

                r_cp(nxt, bn).start()

    for b in range(NBUF_S):
        jlast = NCHUNK - NBUF_S + ((b - NCHUNK) % NBUF_S)
        s_cp(jlast, b).wait()

    plsc.subcore_barrier()
    pltpu.sync_copy(acc.at[pl.ds(row0, ROWS_PER_TILE)],
                    out_hbm.at[cid, pl.ds(row0, ROWS_PER_TILE)])


def kernel(nodes_x, nodes_y, x, directed_edges, weights, W1, b1, W2, b2):
    xT = jnp.transpose(x[0], (1, 0))
    pad = jnp.zeros((N, CW - ND), jnp.float32)
    t1 = jnp.concatenate([nodes_x[0], pad], axis=1)
    t2 = jnp.concatenate([nodes_y[0], pad], axis=1)

    src = directed_edges[0, :, 0].reshape(NW, NCHUNK, CH)
    tgt = directed_edges[0, :, 1].reshape(NW, NCHUNK, CH)

    wpad = jnp.zeros((CW - ND, FC), jnp.float32)
    w1a = jnp.concatenate([W1[:ND], wpad], axis=0)
    w1b = jnp.concatenate([W1[ND:], wpad], axis=0)

    gx = _gather_x(xT, src)
    g1, g2 = _gather_coords(t1, t2, src, tgt)
    feat = _mlp_stage(g1, g2, gx, weights[0].reshape(E, 1), w1a, w1b, b1, W2, b2)
    partial = _scatter_stage(feat, tgt)
    out = partial[0, :N] + partial[1, :N]
    return jnp.transpose(out, (1, 0))[None]


# speedup vs baseline: 22.4229x; 1.2559x over previous
import functools

import jax
import jax.numpy as jnp
import numpy as np
from jax import lax
from jax.experimental import pallas as pl
from jax.experimental.pallas import tpu as pltpu
from jax.experimental.pallas import tpu_sc as plsc

N = 10000
E = 320000
C = 128
ND = 3
FC = 128

NC, NS = 2, 16
NW = NC * NS
EW = E // NW
CH = 80
NCHUNK = EW // CH

NP = 10240
ROWS_PER_TILE = NP // NS

CW = 16

EB = 3200
GRID = E // EB

_mesh = plsc.VectorSubcoreMesh(core_axis_name="c", subcore_axis_name="s")


NBUF = 6


@functools.partial(
    pl.kernel,
    out_type=jax.ShapeDtypeStruct((E, C), jnp.float32),
    mesh=_mesh,
    scratch_types=(
        (pltpu.VMEM((NCHUNK, CH), jnp.int32),)
        + (pltpu.VMEM((CH, C), jnp.float32),) * NBUF
        + (pltpu.SemaphoreType.DMA,) * (2 * NBUF)
    ),
)
def _gather_x(xt_hbm, src_hbm, gx_hbm, src_v, *bs):
    bufs, gsems, wsems = bs[:NBUF], bs[NBUF:2 * NBUF], bs[2 * NBUF:]
    wid = lax.axis_index("c") * NS + lax.axis_index("s")
    base = wid * EW
    pltpu.sync_copy(src_hbm.at[wid], src_v)

    def g_cp(j, b):
        return pltpu.make_async_copy(xt_hbm.at[src_v.at[j]], bufs[b], gsems[b])

    def w_cp(j, b):
        return pltpu.make_async_copy(
            bufs[b], gx_hbm.at[pl.ds(base + j * CH, CH)], wsems[b])

    for b in range(NBUF - 1):
        g_cp(b, b).start()

    @pl.loop(0, NCHUNK + NBUF - 1 - ((NCHUNK - 1) % NBUF), step=NBUF)
    def _chunks(j):
        for b in range(NBUF):
            jj = j + b
            nxt = jj + NBUF - 1

            @pl.when(jj < NCHUNK)
            def _():
                g_cp(jj, b).wait()
                w_cp(jj, b).start()

            @pl.when(nxt < NCHUNK)
            def _():
                bn = (b + NBUF - 1) % NBUF

                @pl.when(nxt >= NBUF)
                def _():
                    w_cp(nxt - NBUF, bn).wait()

                g_cp(nxt, bn).start()

    for b in range(NBUF):
        jlast = NCHUNK - NBUF + ((b - NCHUNK) % NBUF)
        w_cp(jlast, b).wait()


@functools.partial(
    pl.kernel,
    out_type=(
        jax.ShapeDtypeStruct((E, CW), jnp.float32),
        jax.ShapeDtypeStruct((E, CW), jnp.float32),
    ),
    mesh=_mesh,
    scratch_types=(
        (pltpu.VMEM((NCHUNK, CH), jnp.int32),) * 2
        + (pltpu.VMEM((CH, CW), jnp.float32),) * (2 * NBUF)
        + (pltpu.SemaphoreType.DMA,) * (4 * NBUF)
    ),
    compiler_params=pltpu.CompilerParams(use_tc_tiling_on_sc=False),
)
def _gather_coords(t1_hbm, t2_hbm, src_hbm, tgt_hbm, g1_hbm, g2_hbm,
                   src_v, tgt_v, *bs):
    buf1, buf2 = bs[:NBUF], bs[NBUF:2 * NBUF]
    gs1 = bs[2 * NBUF:3 * NBUF]
    gs2 = bs[3 * NBUF:4 * NBUF]
    ws1 = bs[4 * NBUF:5 * NBUF]
    ws2 = bs[5 * NBUF:6 * NBUF]
    wid = lax.axis_index("c") * NS + lax.axis_index("s")
    base = wid * EW
    pltpu.sync_copy(src_hbm.at[wid], src_v)
    pltpu.sync_copy(tgt_hbm.at[wid], tgt_v)

    def g_cp(j, b):
        return (pltpu.make_async_copy(t1_hbm.at[src_v.at[j]], buf1[b], gs1[b]),
                pltpu.make_async_copy(t2_hbm.at[tgt_v.at[j]], buf2[b], gs2[b]))

    def w_cp(j, b):
        eb = base + j * CH
        return (pltpu.make_async_copy(buf1[b], g1_hbm.at[pl.ds(eb, CH)], ws1[b]),
                pltpu.make_async_copy(buf2[b], g2_hbm.at[pl.ds(eb, CH)], ws2[b]))

    for b in range(NBUF - 1):
        for cp in g_cp(b, b):
            cp.start()

    @pl.loop(0, NCHUNK + NBUF - 1 - ((NCHUNK - 1) % NBUF), step=NBUF)
    def _chunks(j):
        for b in range(NBUF):
            jj = j + b
            nxt = jj + NBUF - 1

            @pl.when(jj < NCHUNK)
            def _():
                for cp in g_cp(jj, b):
                    cp.wait()
                for cp in w_cp(jj, b):
                    cp.start()

            @pl.when(nxt < NCHUNK)
            def _():
                bn = (b + NBUF - 1) % NBUF

                @pl.when(nxt >= NBUF)
                def _():
                    for cp in w_cp(nxt - NBUF, bn):
                        cp.wait()

                for cp in g_cp(nxt, bn):
                    cp.start()

    for b in range(NBUF):
        jlast = NCHUNK - NBUF + ((b - NCHUNK) % NBUF)
        for cp in w_cp(jlast, b):
            cp.wait()


def _mlp_body(g1_ref, g2_ref, gx_ref, w_ref, w1a_ref, w1b_ref, b1_ref,
              w2_ref, b2_ref, out_ref):
    cp1 = g1_ref[...]
    cp2 = g2_ref[...]
    g1 = jnp.concatenate([cp1[:, k * CW:(k + 1) * CW] for k in range(C // CW)],
                         axis=0)
    g2 = jnp.concatenate([cp2[:, k * CW:(k + 1) * CW] for k in range(C // CW)],
                         axis=0)
    h = jnp.dot(g1, w1a_ref[...], preferred_element_type=jnp.float32)
    h = h + jnp.dot(g2, w1b_ref[...], preferred_element_type=jnp.float32)
    h = h + b1_ref[...][None, :]
    h = 0.5 * h * (1.0 + lax.erf(h * 0.7071067811865476))
    ker = jnp.dot(h, w2_ref[...], preferred_element_type=jnp.float32)
    ker = ker + b2_ref[...][None, :]
    out_ref[...] = ker * (w_ref[...] * gx_ref[...])


def _mlp_stage(g1, g2, gx, w, w1a, w1b, b1, W2, b2):
    return pl.pallas_call(
        _mlp_body,
        grid=(GRID,),
        in_specs=[
            pl.BlockSpec((EB * CW // C, C), lambda i: (i, 0)),
            pl.BlockSpec((EB * CW // C, C), lambda i: (i, 0)),
            pl.BlockSpec((EB, C), lambda i: (i, 0)),
            pl.BlockSpec((EB, 1), lambda i: (i, 0)),
            pl.BlockSpec((CW, FC), lambda i: (0, 0)),
            pl.BlockSpec((CW, FC), lambda i: (0, 0)),
            pl.BlockSpec((FC,), lambda i: (0,)),
            pl.BlockSpec((FC, C), lambda i: (0, 0)),
            pl.BlockSpec((C,), lambda i: (0,)),
        ],
        out_specs=pl.BlockSpec((EB, C), lambda i: (i, 0)),
        out_shape=jax.ShapeDtypeStruct((E, C), jnp.float32),
        compiler_params=pltpu.CompilerParams(
            dimension_semantics=("arbitrary",),
        ),
    )(g1, g2, gx, w, w1a, w1b, b1, W2, b2)


NBUF_S = 3

@functools.partial(
    pl.kernel,
    out_type=jax.ShapeDtypeStruct((NC, NP, C), jnp.float32),
    mesh=_mesh,
    scratch_types=(
        (pltpu.VMEM((NCHUNK, CH), jnp.int32),
         pltpu.VMEM((16, C), jnp.float32),
         pltpu.VMEM_SHARED((NP, C), jnp.float32),
         pltpu.SemaphoreType.DMA)
        + (pltpu.VMEM((CH, C), jnp.float32),) * NBUF_S
        + (pltpu.SemaphoreType.DMA,) * (2 * NBUF_S)
    ),
)
def _scatter_stage(feat_hbm, tgt_hbm, out_hbm, tgt_v, zbuf, acc, zsem, *bs):
    fbuf, rsems, ssems = bs[:NBUF_S], bs[NBUF_S:2 * NBUF_S], bs[2 * NBUF_S:]
    cid = lax.axis_index("c")
    sid = lax.axis_index("s")
    wid = cid * NS + sid

    for r in range(16):
        for cc in range(C // 16):
            zbuf[r, pl.ds(cc * 16, 16)] = jnp.zeros((16,), jnp.float32)
    row0 = sid * ROWS_PER_TILE

    @pl.loop(0, ROWS_PER_TILE // 16)
    def _zero(j):
        pltpu.async_copy(zbuf, acc.at[pl.ds(row0 + j * 16, 16)], zsem)

    @pl.loop(0, ROWS_PER_TILE // 16)
    def _zdrain(j):
        pltpu.make_async_copy(zbuf, acc.at[pl.ds(row0 + j * 16, 16)],
                              zsem).wait()

    plsc.subcore_barrier()

    pltpu.sync_copy(tgt_hbm.at[wid], tgt_v)
    base = wid * EW

    def r_cp(j, b):
        return pltpu.make_async_copy(
            feat_hbm.at[pl.ds(base + j * CH, CH)], fbuf[b], rsems[b])

    def s_cp(j, b):
        return pltpu.make_async_copy(fbuf[b], acc.at[tgt_v.at[j]], ssems[b])

    for b in range(NBUF_S - 1):
        r_cp(b, b).start()

    @pl.loop(0, NCHUNK + NBUF_S - 1 - ((NCHUNK - 1) % NBUF_S), step=NBUF_S)
    def _chunks(j):
        for b in range(NBUF_S):
            jj = j + b
            nxt = jj + NBUF_S - 1

            @pl.when(jj < NCHUNK)
            def _():
                r_cp(jj, b).wait()
                pltpu.async_copy(fbuf[b], acc.at[tgt_v.at[jj]], ssems[b],
                                 add=True)

            @pl.when(nxt < NCHUNK)
            def _():
                bn = (b + NBUF_S - 1) % NBUF_S

                @pl.when(nxt >= NBUF_S)
                def _():
                    s_cp(nxt - NBUF_S, bn).wait()

                r_cp(nxt, bn).start()

    for b in range(NBUF_S):
        jlast = NCHUNK - NBUF_S + ((b - NCHUNK) % NBUF_S)
        s_cp(jlast, b).wait()

    plsc.subcore_barrier()
    pltpu.sync_copy(acc.at[pl.ds(row0, ROWS_PER_TILE)],
                    out_hbm.at[cid, pl.ds(row0, ROWS_PER_TILE)])


def kernel(nodes_x, nodes_y, x, directed_edges, weights, W1, b1, W2, b2):
    xT = jnp.transpose(x[0], (1, 0))
    pad = jnp.zeros((N, CW - ND), jnp.float32)
    t1 = jnp.concatenate([nodes_x[0], pad], axis=1)
    t2 = jnp.concatenate([nodes_y[0], pad], axis=1)

    src = directed_edges[0, :, 0]
    tgt = directed_edges[0, :, 1]

    def kmajor(a):
        return a.reshape(E // EB, EB // 8, 8).swapaxes(1, 2).reshape(E)

    src_c = src.reshape(NW, NCHUNK, CH)
    tgt_c = tgt.reshape(NW, NCHUNK, CH)
    srcP = kmajor(src).reshape(NW, NCHUNK, CH)
    tgtP = kmajor(tgt).reshape(NW, NCHUNK, CH)
    wP = kmajor(weights[0]).reshape(E, 1)

    wpad = jnp.zeros((CW - ND, FC), jnp.float32)
    w1a = jnp.concatenate([W1[:ND], wpad], axis=0)
    w1b = jnp.concatenate([W1[ND:], wpad], axis=0)

    gx = _gather_x(xT, srcP)
    g1, g2 = _gather_coords(t1, t2, src_c, tgt_c)
    g1 = g1.reshape(E * CW // C, C)
    g2 = g2.reshape(E * CW // C, C)
    feat = _mlp_stage(g1, g2, gx, wP, w1a, w1b, b1, W2, b2)
    partial = _scatter_stage(feat, tgtP)
    out = partial[0, :N] + partial[1, :N]
    return jnp.transpose(out, (1, 0))[None]


# speedup vs baseline: 26.5331x; 1.1833x over previous
import functools

import jax
import jax.numpy as jnp
import numpy as np
from jax import lax
from jax.experimental import pallas as pl
from jax.experimental.pallas import tpu as pltpu
from jax.experimental.pallas import tpu_sc as plsc

N = 10000
E = 320000
C = 128
ND = 3
FC = 128

NC, NS = 2, 16
NW = NC * NS
EW = E // NW
CH = 80
NCHUNK = EW // CH

NP = 10240
ROWS_PER_TILE = NP // NS

CW = 16

EB = 6400
GRID = E // EB

_mesh = plsc.VectorSubcoreMesh(core_axis_name="c", subcore_axis_name="s")


NBUF = 6


@functools.partial(
    pl.kernel,
    out_type=jax.ShapeDtypeStruct((E, C), jnp.float32),
    mesh=_mesh,
    scratch_types=(
        (pltpu.VMEM((NCHUNK, CH), jnp.int32),)
        + (pltpu.VMEM((CH, C), jnp.float32),) * NBUF
        + (pltpu.SemaphoreType.DMA,) * (2 * NBUF)
    ),
)
def _gather_x(xt_hbm, src_hbm, gx_hbm, src_v, *bs):
    bufs, gsems, wsems = bs[:NBUF], bs[NBUF:2 * NBUF], bs[2 * NBUF:]
    wid = lax.axis_index("c") * NS + lax.axis_index("s")
    base = wid * EW
    pltpu.sync_copy(src_hbm.at[wid], src_v)

    def g_cp(j, b):
        return pltpu.make_async_copy(xt_hbm.at[src_v.at[j]], bufs[b], gsems[b])

    def w_cp(j, b):
        return pltpu.make_async_copy(
            bufs[b], gx_hbm.at[pl.ds(base + j * CH, CH)], wsems[b])

    for b in range(NBUF - 1):
        g_cp(b, b).start()

    @pl.loop(0, NCHUNK + NBUF - 1 - ((NCHUNK - 1) % NBUF), step=NBUF)
    def _chunks(j):
        for b in range(NBUF):
            jj = j + b
            nxt = jj + NBUF - 1

            @pl.when(jj < NCHUNK)
            def _():
                g_cp(jj, b).wait()
                w_cp(jj, b).start()

            @pl.when(nxt < NCHUNK)
            def _():
                bn = (b + NBUF - 1) % NBUF

                @pl.when(nxt >= NBUF)
                def _():
                    w_cp(nxt - NBUF, bn).wait()

                g_cp(nxt, bn).start()

    for b in range(NBUF):
        jlast = NCHUNK - NBUF + ((b - NCHUNK) % NBUF)
        w_cp(jlast, b).wait()


RPB = EB // 8
CPB = EB // CH


@functools.partial(
    pl.kernel,
    out_type=(
        jax.ShapeDtypeStruct((E, CW), jnp.float32),
        jax.ShapeDtypeStruct((E, CW), jnp.float32),
    ),
    mesh=_mesh,
    scratch_types=(
        (pltpu.VMEM((EB,), jnp.int32),) * 2
        + (pltpu.VMEM((CPB * CH,), jnp.int32),)
        + (pltpu.VMEM((CH,), jnp.int32),) * (2 * NBUF)
        + (pltpu.VMEM((CH, CW), jnp.float32),) * (2 * NBUF)
        + (pltpu.SemaphoreType.DMA,) * (4 * NBUF)
    ),
    compiler_params=pltpu.CompilerParams(use_tc_tiling_on_sc=False,
                                         needs_layout_passes=False),
)
def _gather_coords(t1_hbm, t2_hbm, srcf_hbm, tgtf_hbm, qtab_hbm,
                   g1_hbm, g2_hbm, blk1, blk2, qtab_v, *bs):
    idx1, idx2 = bs[:NBUF], bs[NBUF:2 * NBUF]
    buf1, buf2 = bs[2 * NBUF:3 * NBUF], bs[3 * NBUF:4 * NBUF]
    gs1 = bs[4 * NBUF:5 * NBUF]
    gs2 = bs[5 * NBUF:6 * NBUF]
    ws1 = bs[6 * NBUF:7 * NBUF]
    ws2 = bs[7 * NBUF:8 * NBUF]
    wid = lax.axis_index("c") * NS + lax.axis_index("s")
    base = wid * EW
    pltpu.sync_copy(qtab_hbm, qtab_v)

    def extract(j, b, force_load=False):
        cg = wid * NCHUNK + j
        blk = cg // CPB
        cpp = cg % CPB

        def _load():
            pltpu.sync_copy(srcf_hbm.at[pl.ds(blk * EB, EB)], blk1)
            pltpu.sync_copy(tgtf_hbm.at[pl.ds(blk * EB, EB)], blk2)

        if force_load:
            _load()
        else:
            pl.when(cpp == 0)(_load)

        for g in range(CH // 16):
            qv = qtab_v[pl.ds(cpp * CH + 16 * g, 16)]
            idx1[b][pl.ds(16 * g, 16)] = plsc.load_gather(blk1, [qv])
            idx2[b][pl.ds(16 * g, 16)] = plsc.load_gather(blk2, [qv])

    def g_cp(j, b):
        return (pltpu.make_async_copy(t1_hbm.at[idx1[b]], buf1[b], gs1[b]),
                pltpu.make_async_copy(t2_hbm.at[idx2[b]], buf2[b], gs2[b]))

    def w_cp(j, b):
        eb = base + j * CH
        return (pltpu.make_async_copy(buf1[b], g1_hbm.at[pl.ds(eb, CH)], ws1[b]),
                pltpu.make_async_copy(buf2[b], g2_hbm.at[pl.ds(eb, CH)], ws2[b]))

    for b in range(NBUF - 1):
        extract(b, b, force_load=(b == 0))
        for cp in g_cp(b, b):
            cp.start()

    @pl.loop(0, NCHUNK + NBUF - 1 - ((NCHUNK - 1) % NBUF), step=NBUF)
    def _chunks(j):
        for b in range(NBUF):
            jj = j + b
            nxt = jj + NBUF - 1

            @pl.when(jj < NCHUNK)
            def _():
                for cp in g_cp(jj, b):
                    cp.wait()
                for cp in w_cp(jj, b):
                    cp.start()

            @pl.when(nxt < NCHUNK)
            def _():
                bn = (b + NBUF - 1) % NBUF

                @pl.when(nxt >= NBUF)
                def _():
                    for cp in w_cp(nxt - NBUF, bn):
                        cp.wait()

                extract(nxt, bn)
                for cp in g_cp(nxt, bn):
                    cp.start()

    for b in range(NBUF):
        jlast = NCHUNK - NBUF + ((b - NCHUNK) % NBUF)
        for cp in w_cp(jlast, b):
            cp.wait()


def _mlp_body(g1_ref, g2_ref, gx_ref, w_ref, w1a_ref, w1b_ref, b1_ref,
              w2_ref, b2_ref, out_ref):
    cp1 = g1_ref[...]
    cp2 = g2_ref[...]
    g1 = jnp.concatenate([cp1[:, k * CW:(k + 1) * CW] for k in range(C // CW)],
                         axis=0)
    g2 = jnp.concatenate([cp2[:, k * CW:(k + 1) * CW] for k in range(C // CW)],
                         axis=0)
    h = jnp.dot(g1, w1a_ref[...], preferred_element_type=jnp.float32)
    h = h + jnp.dot(g2, w1b_ref[...], preferred_element_type=jnp.float32)
    h = h + b1_ref[...][None, :]
    h = 0.5 * h * (1.0 + lax.erf(h * 0.7071067811865476))
    ker = jnp.dot(h, w2_ref[...], preferred_element_type=jnp.float32)
    ker = ker + b2_ref[...][None, :]
    out_ref[...] = ker * (w_ref[...] * gx_ref[...])


def _mlp_stage(g1, g2, gx, w, w1a, w1b, b1, W2, b2):
    return pl.pallas_call(
        _mlp_body,
        grid=(GRID,),
        in_specs=[
            pl.BlockSpec((EB * CW // C, C), lambda i: (i, 0)),
            pl.BlockSpec((EB * CW // C, C), lambda i: (i, 0)),
            pl.BlockSpec((EB, C), lambda i: (i, 0)),
            pl.BlockSpec((EB, 1), lambda i: (i, 0)),
            pl.BlockSpec((CW, FC), lambda i: (0, 0)),
            pl.BlockSpec((CW, FC), lambda i: (0, 0)),
            pl.BlockSpec((FC,), lambda i: (0,)),
            pl.BlockSpec((FC, C), lambda i: (0, 0)),
            pl.BlockSpec((C,), lambda i: (0,)),
        ],
        out_specs=pl.BlockSpec((EB, C), lambda i: (i, 0)),
        out_shape=jax.ShapeDtypeStruct((E, C), jnp.float32),
        compiler_params=pltpu.CompilerParams(
            dimension_semantics=("arbitrary",),
        ),
    )(g1, g2, gx, w, w1a, w1b, b1, W2, b2)


NBUF_S = 3

@functools.partial(
    pl.kernel,
    out_type=jax.ShapeDtypeStruct((NC, NP, C), jnp.float32),
    mesh=_mesh,
    scratch_types=(
        (pltpu.VMEM((NCHUNK, CH), jnp.int32),
         pltpu.VMEM((16, C), jnp.float32),
         pltpu.VMEM_SHARED((NP, C), jnp.float32),
         pltpu.SemaphoreType.DMA)
        + (pltpu.VMEM((CH, C), jnp.float32),) * NBUF_S
        + (pltpu.SemaphoreType.DMA,) * (2 * NBUF_S)
    ),
)
def _scatter_stage(feat_hbm, tgt_hbm, out_hbm, tgt_v, zbuf, acc, zsem, *bs):
    fbuf, rsems, ssems = bs[:NBUF_S], bs[NBUF_S:2 * NBUF_S], bs[2 * NBUF_S:]
    cid = lax.axis_index("c")
    sid = lax.axis_index("s")
    wid = cid * NS + sid

    for r in range(16):
        for cc in range(C // 16):
            zbuf[r, pl.ds(cc * 16, 16)] = jnp.zeros((16,), jnp.float32)
    row0 = sid * ROWS_PER_TILE

    @pl.loop(0, ROWS_PER_TILE // 16)
    def _zero(j):
        pltpu.async_copy(zbuf, acc.at[pl.ds(row0 + j * 16, 16)], zsem)

    @pl.loop(0, ROWS_PER_TILE // 16)
    def _zdrain(j):
        pltpu.make_async_copy(zbuf, acc.at[pl.ds(row0 + j * 16, 16)],
                              zsem).wait()

    plsc.subcore_barrier()

    pltpu.sync_copy(tgt_hbm.at[wid], tgt_v)
    base = wid * EW

    def r_cp(j, b):
        return pltpu.make_async_copy(
            feat_hbm.at[pl.ds(base + j * CH, CH)], fbuf[b], rsems[b])

    def s_cp(j, b):
        return pltpu.make_async_copy(fbuf[b], acc.at[tgt_v.at[j]], ssems[b])

    for b in range(NBUF_S - 1):
        r_cp(b, b).start()

    @pl.loop(0, NCHUNK + NBUF_S - 1 - ((NCHUNK - 1) % NBUF_S), step=NBUF_S)
    def _chunks(j):
        for b in range(NBUF_S):
            jj = j + b
            nxt = jj + NBUF_S - 1

            @pl.when(jj < NCHUNK)
            def _():
                r_cp(jj, b).wait()
                pltpu.async_copy(fbuf[b], acc.at[tgt_v.at[jj]], ssems[b],
                                 add=True)

            @pl.when(nxt < NCHUNK)
            def _():
                bn = (b + NBUF_S - 1) % NBUF_S

                @pl.when(nxt >= NBUF_S)
                def _():
                    s_cp(nxt - NBUF_S, bn).wait()

                r_cp(nxt, bn).start()

    for b in range(NBUF_S):
        jlast = NCHUNK - NBUF_S + ((b - NCHUNK) % NBUF_S)
        s_cp(jlast, b).wait()

    plsc.subcore_barrier()
    pltpu.sync_copy(acc.at[pl.ds(row0, ROWS_PER_TILE)],
                    out_hbm.at[cid, pl.ds(row0, ROWS_PER_TILE)])


def kernel(nodes_x, nodes_y, x, directed_edges, weights, W1, b1, W2, b2):
    xT = jnp.transpose(x[0], (1, 0))
    pad = jnp.zeros((N, CW - ND), jnp.float32)
    t1 = jnp.concatenate([nodes_x[0], pad], axis=1)
    t2 = jnp.concatenate([nodes_y[0], pad], axis=1)

    src = directed_edges[0, :, 0]
    tgt = directed_edges[0, :, 1]
    srcW = src.reshape(NW, NCHUNK, CH)
    tgtW = tgt.reshape(NW, NCHUNK, CH)
    tq, cq = np.meshgrid(np.arange(CH), np.arange(CPB))
    qtab = jnp.asarray(
        (RPB * (tq % 8) + (CH // 8) * cq + tq // 8).reshape(-1).astype(np.int32))

    wpad = jnp.zeros((CW - ND, FC), jnp.float32)
    w1a = jnp.concatenate([W1[:ND], wpad], axis=0)
    w1b = jnp.concatenate([W1[ND:], wpad], axis=0)

    gx = _gather_x(xT, srcW)
    g1, g2 = _gather_coords(t1, t2, src, tgt, qtab)
    g1 = g1.reshape(E * CW // C, C)
    g2 = g2.reshape(E * CW // C, C)
    feat = _mlp_stage(g1, g2, gx, weights[0].reshape(E, 1), w1a, w1b, b1, W2, b2)
    partial = _scatter_stage(feat, tgtW)
    out = partial[0, :N] + partial[1, :N]
    return jnp.transpose(out, (1, 0))[None]


# speedup vs baseline: 29.5627x; 1.1142x over previous
import functools

import jax
import jax.numpy as jnp
import numpy as np
from jax import lax
from jax.experimental import pallas as pl
from jax.experimental.pallas import tpu as pltpu
from jax.experimental.pallas import tpu_sc as plsc

N = 10000
E = 320000
C = 128
ND = 3
FC = 128

NC, NS = 2, 16
NW = NC * NS
EW = E // NW
CH = 80
NCHUNK = EW // CH

NP = 10240
ROWS_PER_TILE = NP // NS

CW = 16

EB = 6400
GRID = E // EB

_mesh = plsc.VectorSubcoreMesh(core_axis_name="c", subcore_axis_name="s")


NBUF = 6


@functools.partial(
    pl.kernel,
    out_type=jax.ShapeDtypeStruct((E, C), jnp.float32),
    mesh=_mesh,
    scratch_types=(
        (pltpu.VMEM((NCHUNK, CH), jnp.int32),)
        + (pltpu.VMEM((CH, C), jnp.float32),) * NBUF
        + (pltpu.SemaphoreType.DMA,) * (2 * NBUF)
    ),
)
def _gather_x(xt_hbm, src_hbm, gx_hbm, src_v, *bs):
    bufs, gsems, wsems = bs[:NBUF], bs[NBUF:2 * NBUF], bs[2 * NBUF:]
    wid = lax.axis_index("c") * NS + lax.axis_index("s")
    base = wid * EW
    pltpu.sync_copy(src_hbm.at[wid], src_v)

    def g_cp(j, b):
        return pltpu.make_async_copy(xt_hbm.at[src_v.at[j]], bufs[b], gsems[b])

    def w_cp(j, b):
        return pltpu.make_async_copy(
            bufs[b], gx_hbm.at[pl.ds(base + j * CH, CH)], wsems[b])

    for b in range(NBUF - 1):
        g_cp(b, b).start()

    @pl.loop(0, NCHUNK + NBUF - 1 - ((NCHUNK - 1) % NBUF), step=NBUF)
    def _chunks(j):
        for b in range(NBUF):
            jj = j + b
            nxt = jj + NBUF - 1

            @pl.when(jj < NCHUNK)
            def _():
                g_cp(jj, b).wait()
                w_cp(jj, b).start()

            @pl.when(nxt < NCHUNK)
            def _():
                bn = (b + NBUF - 1) % NBUF

                @pl.when(nxt >= NBUF)
                def _():
                    w_cp(nxt - NBUF, bn).wait()

                g_cp(nxt, bn).start()

    for b in range(NBUF):
        jlast = NCHUNK - NBUF + ((b - NCHUNK) % NBUF)
        w_cp(jlast, b).wait()


RPB = EB // 8
CPB = EB // CH


@functools.partial(
    pl.kernel,
    out_type=(
        jax.ShapeDtypeStruct((E, CW), jnp.float32),
        jax.ShapeDtypeStruct((E, CW), jnp.float32),
    ),
    mesh=_mesh,
    scratch_types=(
        (pltpu.VMEM((EB,), jnp.int32),) * 2
        + (pltpu.VMEM((CPB * CH,), jnp.int32),)
        + (pltpu.VMEM((CH,), jnp.int32),) * (2 * NBUF)
        + (pltpu.VMEM((CH, CW), jnp.float32),) * (2 * NBUF)
        + (pltpu.SemaphoreType.DMA,) * (4 * NBUF)
    ),
    compiler_params=pltpu.CompilerParams(use_tc_tiling_on_sc=False,
                                         needs_layout_passes=False),
)
def _gather_coords(t1_hbm, t2_hbm, srcf_hbm, tgtf_hbm, qtab_hbm,
                   g1_hbm, g2_hbm, blk1, blk2, qtab_v, *bs):
    idx1, idx2 = bs[:NBUF], bs[NBUF:2 * NBUF]
    buf1, buf2 = bs[2 * NBUF:3 * NBUF], bs[3 * NBUF:4 * NBUF]
    gs1 = bs[4 * NBUF:5 * NBUF]
    gs2 = bs[5 * NBUF:6 * NBUF]
    ws1 = bs[6 * NBUF:7 * NBUF]
    ws2 = bs[7 * NBUF:8 * NBUF]
    wid = lax.axis_index("c") * NS + lax.axis_index("s")
    base = wid * EW
    pltpu.sync_copy(qtab_hbm, qtab_v)

    def extract(j, b, force_load=False):
        cg = wid * NCHUNK + j
        blk = cg // CPB
        cpp = cg % CPB

        def _load():
            pltpu.sync_copy(srcf_hbm.at[pl.ds(blk * EB, EB)], blk1)
            pltpu.sync_copy(tgtf_hbm.at[pl.ds(blk * EB, EB)], blk2)

        if force_load:
            _load()
        else:
            pl.when(cpp == 0)(_load)

        for g in range(CH // 16):
            qv = qtab_v[pl.ds(cpp * CH + 16 * g, 16)]
            idx1[b][pl.ds(16 * g, 16)] = plsc.load_gather(blk1, [qv])
            idx2[b][pl.ds(16 * g, 16)] = plsc.load_gather(blk2, [qv])

    def g_cp(j, b):
        return (pltpu.make_async_copy(t1_hbm.at[idx1[b]], buf1[b], gs1[b]),
                pltpu.make_async_copy(t2_hbm.at[idx2[b]], buf2[b], gs2[b]))

    def w_cp(j, b):
        eb = base + j * CH
        return (pltpu.make_async_copy(buf1[b], g1_hbm.at[pl.ds(eb, CH)], ws1[b]),
                pltpu.make_async_copy(buf2[b], g2_hbm.at[pl.ds(eb, CH)], ws2[b]))

    for b in range(NBUF - 1):
        extract(b, b, force_load=(b == 0))
        for cp in g_cp(b, b):
            cp.start()

    @pl.loop(0, NCHUNK + NBUF - 1 - ((NCHUNK - 1) % NBUF), step=NBUF)
    def _chunks(j):
        for b in range(NBUF):
            jj = j + b
            nxt = jj + NBUF - 1

            @pl.when(jj < NCHUNK)
            def _():
                for cp in g_cp(jj, b):
                    cp.wait()
                for cp in w_cp(jj, b):
                    cp.start()

            @pl.when(nxt < NCHUNK)
            def _():
                bn = (b + NBUF - 1) % NBUF

                @pl.when(nxt >= NBUF)
                def _():
                    for cp in w_cp(nxt - NBUF, bn):
                        cp.wait()

                extract(nxt, bn)
                for cp in g_cp(nxt, bn):
                    cp.start()

    for b in range(NBUF):
        jlast = NCHUNK - NBUF + ((b - NCHUNK) % NBUF)
        for cp in w_cp(jlast, b):
            cp.wait()


def _mlp_body(g1_ref, g2_ref, gx_ref, w_ref, w1a_ref, w1b_ref, b1_ref,
              w2_ref, b2_ref, out_ref):
    cp1 = g1_ref[...]
    cp2 = g2_ref[...]
    g1 = jnp.concatenate([cp1[:, k * CW:(k + 1) * CW] for k in range(C // CW)],
                         axis=0)
    g2 = jnp.concatenate([cp2[:, k * CW:(k + 1) * CW] for k in range(C // CW)],
                         axis=0)
    h = jnp.dot(g1, w1a_ref[...], preferred_element_type=jnp.float32)
    h = h + jnp.dot(g2, w1b_ref[...], preferred_element_type=jnp.float32)
    h = h + b1_ref[...][None, :]
    h = 0.5 * h * (1.0 + lax.erf(h * 0.7071067811865476))
    ker = jnp.dot(h, w2_ref[...], preferred_element_type=jnp.float32)
    ker = ker + b2_ref[...][None, :]
    w = jnp.transpose(w_ref[0], (1, 0))
    out_ref[...] = ker * (w * gx_ref[...])


def _mlp_stage(g1, g2, gx, w, w1a, w1b, b1, W2, b2):
    return pl.pallas_call(
        _mlp_body,
        grid=(GRID,),
        in_specs=[
            pl.BlockSpec((EB * CW // C, C), lambda i: (i, 0)),
            pl.BlockSpec((EB * CW // C, C), lambda i: (i, 0)),
            pl.BlockSpec((EB, C), lambda i: (i, 0)),
            pl.BlockSpec((1, 1, EB), lambda i: (i, 0, 0)),
            pl.BlockSpec((CW, FC), lambda i: (0, 0)),
            pl.BlockSpec((CW, FC), lambda i: (0, 0)),
            pl.BlockSpec((FC,), lambda i: (0,)),
            pl.BlockSpec((FC, C), lambda i: (0, 0)),
            pl.BlockSpec((C,), lambda i: (0,)),
        ],
        out_specs=pl.BlockSpec((EB, C), lambda i: (i, 0)),
        out_shape=jax.ShapeDtypeStruct((E, C), jnp.float32),
        compiler_params=pltpu.CompilerParams(
            dimension_semantics=("arbitrary",),
        ),
    )(g1, g2, gx, w, w1a, w1b, b1, W2, b2)


NBUF_S = 3

@functools.partial(
    pl.kernel,
    out_type=jax.ShapeDtypeStruct((NC, NP, C), jnp.float32),
    mesh=_mesh,
    scratch_types=(
        (pltpu.VMEM((NCHUNK, CH), jnp.int32),
         pltpu.VMEM((16, C), jnp.float32),
         pltpu.VMEM_SHARED((NP, C), jnp.float32),
         pltpu.SemaphoreType.DMA)
        + (pltpu.VMEM((CH, C), jnp.float32),) * NBUF_S
        + (pltpu.SemaphoreType.DMA,) * (2 * NBUF_S)
    ),
)
def _scatter_stage(feat_hbm, tgt_hbm, out_hbm, tgt_v, zbuf, acc, zsem, *bs):
    fbuf, rsems, ssems = bs[:NBUF_S], bs[NBUF_S:2 * NBUF_S], bs[2 * NBUF_S:]
    cid = lax.axis_index("c")
    sid = lax.axis_index("s")
    wid = cid * NS + sid

    for r in range(16):
        for cc in range(C // 16):
            zbuf[r, pl.ds(cc * 16, 16)] = jnp.zeros((16,), jnp.float32)
    row0 = sid * ROWS_PER_TILE

    @pl.loop(0, ROWS_PER_TILE // 16)
    def _zero(j):
        pltpu.async_copy(zbuf, acc.at[pl.ds(row0 + j * 16, 16)], zsem)

    @pl.loop(0, ROWS_PER_TILE // 16)
    def _zdrain(j):
        pltpu.make_async_copy(zbuf, acc.at[pl.ds(row0 + j * 16, 16)],
                              zsem).wait()

    plsc.subcore_barrier()

    pltpu.sync_copy(tgt_hbm.at[wid], tgt_v)
    base = wid * EW

    def r_cp(j, b):
        return pltpu.make_async_copy(
            feat_hbm.at[pl.ds(base + j * CH, CH)], fbuf[b], rsems[b])

    def s_cp(j, b):
        return pltpu.make_async_copy(fbuf[b], acc.at[tgt_v.at[j]], ssems[b])

    for b in range(NBUF_S - 1):
        r_cp(b, b).start()

    @pl.loop(0, NCHUNK + NBUF_S - 1 - ((NCHUNK - 1) % NBUF_S), step=NBUF_S)
    def _chunks(j):
        for b in range(NBUF_S):
            jj = j + b
            nxt = jj + NBUF_S - 1

            @pl.when(jj < NCHUNK)
            def _():
                r_cp(jj, b).wait()
                pltpu.async_copy(fbuf[b], acc.at[tgt_v.at[jj]], ssems[b],
                                 add=True)

            @pl.when(nxt < NCHUNK)
            def _():
                bn = (b + NBUF_S - 1) % NBUF_S

                @pl.when(nxt >= NBUF_S)
                def _():
                    s_cp(nxt - NBUF_S, bn).wait()

                r_cp(nxt, bn).start()

    for b in range(NBUF_S):
        jlast = NCHUNK - NBUF_S + ((b - NCHUNK) % NBUF_S)
        s_cp(jlast, b).wait()

    plsc.subcore_barrier()
    pltpu.sync_copy(acc.at[pl.ds(row0, ROWS_PER_TILE)],
                    out_hbm.at[cid, pl.ds(row0, ROWS_PER_TILE)])


def kernel(nodes_x, nodes_y, x, directed_edges, weights, W1, b1, W2, b2):
    xT = jnp.transpose(x[0], (1, 0))
    pad = jnp.zeros((N, CW - ND), jnp.float32)
    t1 = jnp.concatenate([nodes_x[0], pad], axis=1)
    t2 = jnp.concatenate([nodes_y[0], pad], axis=1)

    src = directed_edges[0, :, 0]
    tgt = directed_edges[0, :, 1]
    srcW = src.reshape(NW, NCHUNK, CH)
    tgtW = tgt.reshape(NW, NCHUNK, CH)
    tq, cq = np.meshgrid(np.arange(CH), np.arange(CPB))
    qtab = jnp.asarray(
        (RPB * (tq % 8) + (CH // 8) * cq + tq // 8).reshape(-1).astype(np.int32))

    wpad = jnp.zeros((CW - ND, FC), jnp.float32)
    w1a = jnp.concatenate([W1[:ND], wpad], axis=0)
    w1b = jnp.concatenate([W1[ND:], wpad], axis=0)

    gx = _gather_x(xT, srcW)
    g1, g2 = _gather_coords(t1, t2, src, tgt, qtab)
    g1 = g1.reshape(E * CW // C, C)
    g2 = g2.reshape(E * CW // C, C)
    feat = _mlp_stage(g1, g2, gx, weights[0].reshape(GRID, 1, EB), w1a, w1b,
                      b1, W2, b2)
    partial = _scatter_stage(feat, tgtW)
    out = partial[0, :N] + partial[1, :N]
    return jnp.transpose(out, (1, 0))[None]


# speedup vs baseline: 32.7409x; 1.1075x over previous
import functools

import jax
import jax.numpy as jnp
import numpy as np
from jax import lax
from jax.experimental import pallas as pl
from jax.experimental.pallas import tpu as pltpu
from jax.experimental.pallas import tpu_sc as plsc

N = 10000
E = 320000
C = 128
ND = 3
FC = 128

NC, NS = 2, 16
NW = NC * NS
EW = E // NW
CH = 80
NCHUNK = EW // CH

NP = 10240
ROWS_PER_TILE = NP // NS

CW = 16

EB = 6400
GRID = E // EB

_mesh = plsc.VectorSubcoreMesh(core_axis_name="c", subcore_axis_name="s")


NBUF = 6


NBX = 4
NROWS_TILE = NP // NS


@functools.partial(
    pl.kernel,
    out_type=jax.ShapeDtypeStruct((E, C), jnp.float32),
    mesh=_mesh,
    scratch_types=(
        (pltpu.VMEM_SHARED((NP, C), jnp.float32),)
        + (pltpu.VMEM((CH,), jnp.int32),) * NBX
        + (pltpu.VMEM((CH, C), jnp.float32),) * NBX
        + (pltpu.SemaphoreType.DMA,) * (3 * NBX)
    ),
)
def _gather_x(xt_hbm, src_hbm, gx_hbm, xs, *bs):
    idxs, bufs = bs[:NBX], bs[NBX:2 * NBX]
    isems = bs[2 * NBX:3 * NBX]
    gsems = bs[3 * NBX:4 * NBX]
    wsems = bs[4 * NBX:5 * NBX]
    cid = lax.axis_index("c")
    sid = lax.axis_index("s")
    wid = cid * NS + sid
    base = wid * EW

    r0 = sid * NROWS_TILE
    pltpu.sync_copy(xt_hbm.at[pl.ds(r0, NROWS_TILE)],
                    xs.at[pl.ds(r0, NROWS_TILE)])
    plsc.subcore_barrier()

    def i_cp(j, b):
        return pltpu.make_async_copy(src_hbm.at[wid, j], idxs[b], isems[b])

    def g_cp(j, b):
        return pltpu.make_async_copy(xs.at[idxs[b]], bufs[b], gsems[b])

    def w_cp(j, b):
        return pltpu.make_async_copy(
            bufs[b], gx_hbm.at[pl.ds(base + j * CH, CH)], wsems[b])

    for b in range(NBX):
        i_cp(b, b).start()
    for b in range(NBX - 1):
        i_cp(b, b).wait()
        g_cp(b, b).start()

    @pl.loop(0, NCHUNK + NBX - 1 - ((NCHUNK - 1) % NBX), step=NBX)
    def _chunks(j):
        for b in range(NBX):
            jj = j + b
            nxt = jj + NBX - 1

            @pl.when(jj < NCHUNK)
            def _():
                g_cp(jj, b).wait()

                @pl.when(jj + NBX < NCHUNK)
                def _():
                    i_cp(jj + NBX, b).start()

                w_cp(jj, b).start()

            @pl.when(nxt < NCHUNK)
            def _():
                bn = (b + NBX - 1) % NBX

                @pl.when(nxt >= NBX)
                def _():
                    w_cp(nxt - NBX, bn).wait()

                i_cp(nxt, bn).wait()
                g_cp(nxt, bn).start()

    for b in range(NBX):
        jlast = NCHUNK - NBX + ((b - NCHUNK) % NBX)
        w_cp(jlast, b).wait()


RPB = EB // 8
CPB = EB // CH


@functools.partial(
    pl.kernel,
    out_type=(
        jax.ShapeDtypeStruct((E, CW), jnp.float32),
        jax.ShapeDtypeStruct((E, CW), jnp.float32),
    ),
    mesh=_mesh,
    scratch_types=(
        (pltpu.VMEM((EB,), jnp.int32),) * 2
        + (pltpu.VMEM((CPB * CH,), jnp.int32),)
        + (pltpu.VMEM((CH,), jnp.int32),) * (2 * NBUF)
        + (pltpu.VMEM((CH, CW), jnp.float32),) * (2 * NBUF)
        + (pltpu.SemaphoreType.DMA,) * (4 * NBUF)
    ),
    compiler_params=pltpu.CompilerParams(use_tc_tiling_on_sc=False,
                                         needs_layout_passes=False),
)
def _gather_coords(t1_hbm, t2_hbm, srcf_hbm, tgtf_hbm, qtab_hbm,
                   g1_hbm, g2_hbm, blk1, blk2, qtab_v, *bs):
    idx1, idx2 = bs[:NBUF], bs[NBUF:2 * NBUF]
    buf1, buf2 = bs[2 * NBUF:3 * NBUF], bs[3 * NBUF:4 * NBUF]
    gs1 = bs[4 * NBUF:5 * NBUF]
    gs2 = bs[5 * NBUF:6 * NBUF]
    ws1 = bs[6 * NBUF:7 * NBUF]
    ws2 = bs[7 * NBUF:8 * NBUF]
    wid = lax.axis_index("c") * NS + lax.axis_index("s")
    base = wid * EW
    pltpu.sync_copy(qtab_hbm, qtab_v)

    def extract(j, b, force_load=False):
        cg = wid * NCHUNK + j
        blk = cg // CPB
        cpp = cg % CPB

        def _load():
            pltpu.sync_copy(srcf_hbm.at[pl.ds(blk * EB, EB)], blk1)
            pltpu.sync_copy(tgtf_hbm.at[pl.ds(blk * EB, EB)], blk2)

        if force_load:
            _load()
        else:
            pl.when(cpp == 0)(_load)

        for g in range(CH // 16):
            qv = qtab_v[pl.ds(cpp * CH + 16 * g, 16)]
            idx1[b][pl.ds(16 * g, 16)] = plsc.load_gather(blk1, [qv])
            idx2[b][pl.ds(16 * g, 16)] = plsc.load_gather(blk2, [qv])

    def g_cp(j, b):
        return (pltpu.make_async_copy(t1_hbm.at[idx1[b]], buf1[b], gs1[b]),
                pltpu.make_async_copy(t2_hbm.at[idx2[b]], buf2[b], gs2[b]))

    def w_cp(j, b):
        eb = base + j * CH
        return (pltpu.make_async_copy(buf1[b], g1_hbm.at[pl.ds(eb, CH)], ws1[b]),
                pltpu.make_async_copy(buf2[b], g2_hbm.at[pl.ds(eb, CH)], ws2[b]))

    for b in range(NBUF - 1):
        extract(b, b, force_load=(b == 0))
        for cp in g_cp(b, b):
            cp.start()

    @pl.loop(0, NCHUNK + NBUF - 1 - ((NCHUNK - 1) % NBUF), step=NBUF)
    def _chunks(j):
        for b in range(NBUF):
            jj = j + b
            nxt = jj + NBUF - 1

            @pl.when(jj < NCHUNK)
            def _():
                for cp in g_cp(jj, b):
                    cp.wait()
                for cp in w_cp(jj, b):
                    cp.start()

            @pl.when(nxt < NCHUNK)
            def _():
                bn = (b + NBUF - 1) % NBUF

                @pl.when(nxt >= NBUF)
                def _():
                    for cp in w_cp(nxt - NBUF, bn):
                        cp.wait()

                extract(nxt, bn)
                for cp in g_cp(nxt, bn):
                    cp.start()

    for b in range(NBUF):
        jlast = NCHUNK - NBUF + ((b - NCHUNK) % NBUF)
        for cp in w_cp(jlast, b):
            cp.wait()


def _mlp_body(g1_ref, g2_ref, gx_ref, w_ref, w1a_ref, w1b_ref, b1_ref,
              w2_ref, b2_ref, out_ref):
    cp1 = g1_ref[...]
    cp2 = g2_ref[...]
    g1 = jnp.concatenate([cp1[:, k * CW:(k + 1) * CW] for k in range(C // CW)],
                         axis=0)
    g2 = jnp.concatenate([cp2[:, k * CW:(k + 1) * CW] for k in range(C // CW)],
                         axis=0)
    h = jnp.dot(g1, w1a_ref[...], preferred_element_type=jnp.float32)
    h = h + jnp.dot(g2, w1b_ref[...], preferred_element_type=jnp.float32)
    h = h + b1_ref[...][None, :]
    h = 0.5 * h * (1.0 + lax.erf(h * 0.7071067811865476))
    ker = jnp.dot(h, w2_ref[...], preferred_element_type=jnp.float32)
    ker = ker + b2_ref[...][None, :]
    w = jnp.transpose(w_ref[0], (1, 0))
    out_ref[...] = ker * (w * gx_ref[...])


def _mlp_stage(g1, g2, gx, w, w1a, w1b, b1, W2, b2):
    return pl.pallas_call(
        _mlp_body,
        grid=(GRID,),
        in_specs=[
            pl.BlockSpec((EB * CW // C, C), lambda i: (i, 0)),
            pl.BlockSpec((EB * CW // C, C), lambda i: (i, 0)),
            pl.BlockSpec((EB, C), lambda i: (i, 0)),
            pl.BlockSpec((1, 1, EB), lambda i: (i, 0, 0)),
            pl.BlockSpec((CW, FC), lambda i: (0, 0)),
            pl.BlockSpec((CW, FC), lambda i: (0, 0)),
            pl.BlockSpec((FC,), lambda i: (0,)),
            pl.BlockSpec((FC, C), lambda i: (0, 0)),
            pl.BlockSpec((C,), lambda i: (0,)),
        ],
        out_specs=pl.BlockSpec((EB, C), lambda i: (i, 0)),
        out_shape=jax.ShapeDtypeStruct((E, C), jnp.float32),
        compiler_params=pltpu.CompilerParams(
            dimension_semantics=("arbitrary",),
        ),
    )(g1, g2, gx, w, w1a, w1b, b1, W2, b2)


NBUF_S = 3

@functools.partial(
    pl.kernel,
    out_type=jax.ShapeDtypeStruct((NC, NP, C), jnp.float32),
    mesh=_mesh,
    scratch_types=(
        (pltpu.VMEM((NCHUNK, CH), jnp.int32),
         pltpu.VMEM((16, C), jnp.float32),
         pltpu.VMEM_SHARED((NP, C), jnp.float32),
         pltpu.SemaphoreType.DMA)
        + (pltpu.VMEM((CH, C), jnp.float32),) * NBUF_S
        + (pltpu.SemaphoreType.DMA,) * (2 * NBUF_S)
    ),
)
def _scatter_stage(feat_hbm, tgt_hbm, out_hbm, tgt_v, zbuf, acc, zsem, *bs):
    fbuf, rsems, ssems = bs[:NBUF_S], bs[NBUF_S:2 * NBUF_S], bs[2 * NBUF_S:]
    cid = lax.axis_index("c")
    sid = lax.axis_index("s")
    wid = cid * NS + sid

    for r in range(16):
        for cc in range(C // 16):
            zbuf[r, pl.ds(cc * 16, 16)] = jnp.zeros((16,), jnp.float32)
    row0 = sid * ROWS_PER_TILE

    @pl.loop(0, ROWS_PER_TILE // 16)
    def _zero(j):
        pltpu.async_copy(zbuf, acc.at[pl.ds(row0 + j * 16, 16)], zsem)

    @pl.loop(0, ROWS_PER_TILE // 16)
    def _zdrain(j):
        pltpu.make_async_copy(zbuf, acc.at[pl.ds(row0 + j * 16, 16)],
                              zsem).wait()

    plsc.subcore_barrier()

    pltpu.sync_copy(tgt_hbm.at[wid], tgt_v)
    base = wid * EW

    def r_cp(j, b):
        return pltpu.make_async_copy(
            feat_hbm.at[pl.ds(base + j * CH, CH)], fbuf[b], rsems[b])

    def s_cp(j, b):
        return pltpu.make_async_copy(fbuf[b], acc.at[tgt_v.at[j]], ssems[b])

    for b in range(NBUF_S - 1):
        r_cp(b, b).start()

    @pl.loop(0, NCHUNK + NBUF_S - 1 - ((NCHUNK - 1) % NBUF_S), step=NBUF_S)
    def _chunks(j):
        for b in range(NBUF_S):
            jj = j + b
            nxt = jj + NBUF_S - 1

            @pl.when(jj < NCHUNK)
            def _():
                r_cp(jj, b).wait()
                pltpu.async_copy(fbuf[b], acc.at[tgt_v.at[jj]], ssems[b],
                                 add=True)

            @pl.when(nxt < NCHUNK)
            def _():
                bn = (b + NBUF_S - 1) % NBUF_S

                @pl.when(nxt >= NBUF_S)
                def _():
                    s_cp(nxt - NBUF_S, bn).wait()

                r_cp(nxt, bn).start()

    for b in range(NBUF_S):
        jlast = NCHUNK - NBUF_S + ((b - NCHUNK) % NBUF_S)
        s_cp(jlast, b).wait()

    plsc.subcore_barrier()
    pltpu.sync_copy(acc.at[pl.ds(row0, ROWS_PER_TILE)],
                    out_hbm.at[cid, pl.ds(row0, ROWS_PER_TILE)])


def kernel(nodes_x, nodes_y, x, directed_edges, weights, W1, b1, W2, b2):
    xT = jnp.transpose(x[0], (1, 0))
    pad = jnp.zeros((N, CW - ND), jnp.float32)
    t1 = jnp.concatenate([nodes_x[0], pad], axis=1)
    t2 = jnp.concatenate([nodes_y[0], pad], axis=1)

    src = directed_edges[0, :, 0]
    tgt = directed_edges[0, :, 1]
    srcW = src.reshape(NW, NCHUNK, CH)
    tgtW = tgt.reshape(NW, NCHUNK, CH)
    tq, cq = np.meshgrid(np.arange(CH), np.arange(CPB))
    qtab = jnp.asarray(
        (RPB * (tq % 8) + (CH // 8) * cq + tq // 8).reshape(-1).astype(np.int32))

    wpad = jnp.zeros((CW - ND, FC), jnp.float32)
    w1a = jnp.concatenate([W1[:ND], wpad], axis=0)
    w1b = jnp.concatenate([W1[ND:], wpad], axis=0)

    xTp = jnp.concatenate([xT, jnp.zeros((NP - N, C), jnp.float32)], axis=0)
    gx = _gather_x(xTp, srcW)
    g1, g2 = _gather_coords(t1, t2, src, tgt, qtab)
    g1 = g1.reshape(E * CW // C, C)
    g2 = g2.reshape(E * CW // C, C)
    feat = _mlp_stage(g1, g2, gx, weights[0].reshape(GRID, 1, EB), w1a, w1b,
                      b1, W2, b2)
    partial = _scatter_stage(feat, tgtW)
    out = partial[0, :N] + partial[1, :N]
    return jnp.transpose(out, (1, 0))[None]


# speedup vs baseline: 33.9146x; 1.0358x over previous
import functools

import jax
import jax.numpy as jnp
import numpy as np
from jax import lax
from jax.experimental import pallas as pl
from jax.experimental.pallas import tpu as pltpu
from jax.experimental.pallas import tpu_sc as plsc

N = 10000
E = 320000
C = 128
ND = 3
FC = 128

NC, NS = 2, 16
NW = NC * NS
EW = E // NW
CH = 80
NCHUNK = EW // CH

NP = 10240
ROWS_PER_TILE = NP // NS

CW = 16

EB = 6400
GRID = E // EB

_mesh = plsc.VectorSubcoreMesh(core_axis_name="c", subcore_axis_name="s")


NBUF = 6


NBX = 4
NROWS_TILE = NP // NS


@functools.partial(
    pl.kernel,
    out_type=jax.ShapeDtypeStruct((E, C), jnp.float32),
    mesh=_mesh,
    scratch_types=(
        (pltpu.VMEM_SHARED((NP, C), jnp.float32),)
        + (pltpu.VMEM((CH,), jnp.int32),) * NBX
        + (pltpu.VMEM((CH, C), jnp.float32),) * NBX
        + (pltpu.SemaphoreType.DMA,) * (3 * NBX)
    ),
)
def _gather_x(xt_hbm, src_hbm, gx_hbm, xs, *bs):
    idxs, bufs = bs[:NBX], bs[NBX:2 * NBX]
    isems = bs[2 * NBX:3 * NBX]
    gsems = bs[3 * NBX:4 * NBX]
    wsems = bs[4 * NBX:5 * NBX]
    cid = lax.axis_index("c")
    sid = lax.axis_index("s")
    wid = cid * NS + sid
    base = wid * EW

    r0 = sid * NROWS_TILE
    pltpu.sync_copy(xt_hbm.at[pl.ds(r0, NROWS_TILE)],
                    xs.at[pl.ds(r0, NROWS_TILE)])
    plsc.subcore_barrier()

    def i_cp(j, b):
        return pltpu.make_async_copy(src_hbm.at[wid, j], idxs[b], isems[b])

    def g_cp(j, b):
        return pltpu.make_async_copy(xs.at[idxs[b]], bufs[b], gsems[b])

    def w_cp(j, b):
        return pltpu.make_async_copy(
            bufs[b], gx_hbm.at[pl.ds(base + j * CH, CH)], wsems[b])

    for b in range(NBX):
        i_cp(b, b).start()
    for b in range(NBX - 1):
        i_cp(b, b).wait()
        g_cp(b, b).start()

    @pl.loop(0, NCHUNK + NBX - 1 - ((NCHUNK - 1) % NBX), step=NBX)
    def _chunks(j):
        for b in range(NBX):
            jj = j + b
            nxt = jj + NBX - 1

            @pl.when(jj < NCHUNK)
            def _():
                g_cp(jj, b).wait()

                @pl.when(jj + NBX < NCHUNK)
                def _():
                    i_cp(jj + NBX, b).start()

                w_cp(jj, b).start()

            @pl.when(nxt < NCHUNK)
            def _():
                bn = (b + NBX - 1) % NBX

                @pl.when(nxt >= NBX)
                def _():
                    w_cp(nxt - NBX, bn).wait()

                i_cp(nxt, bn).wait()
                g_cp(nxt, bn).start()

    for b in range(NBX):
        jlast = NCHUNK - NBX + ((b - NCHUNK) % NBX)
        w_cp(jlast, b).wait()


RPB = EB // 8
CPB = EB // CH


@functools.partial(
    pl.kernel,
    out_type=(
        jax.ShapeDtypeStruct((E, CW), jnp.float32),
        jax.ShapeDtypeStruct((E, CW), jnp.float32),
    ),
    mesh=_mesh,
    scratch_types=(
        (pltpu.VMEM_SHARED((NP, CW), jnp.float32),) * 2
        + (pltpu.VMEM((EB,), jnp.int32),) * 2
        + (pltpu.VMEM((CPB * CH,), jnp.int32),)
        + (pltpu.VMEM((CH,), jnp.int32),) * (2 * NBUF)
        + (pltpu.VMEM((CH, CW), jnp.float32),) * (2 * NBUF)
        + (pltpu.SemaphoreType.DMA,) * (4 * NBUF)
    ),
    compiler_params=pltpu.CompilerParams(use_tc_tiling_on_sc=False,
                                         needs_layout_passes=False),
)
def _gather_coords(t1_hbm, t2_hbm, srcf_hbm, tgtf_hbm, qtab_hbm,
                   g1_hbm, g2_hbm, ts1, ts2, blk1, blk2, qtab_v, *bs):
    idx1, idx2 = bs[:NBUF], bs[NBUF:2 * NBUF]
    buf1, buf2 = bs[2 * NBUF:3 * NBUF], bs[3 * NBUF:4 * NBUF]
    gs1 = bs[4 * NBUF:5 * NBUF]
    gs2 = bs[5 * NBUF:6 * NBUF]
    ws1 = bs[6 * NBUF:7 * NBUF]
    ws2 = bs[7 * NBUF:8 * NBUF]
    sid = lax.axis_index("s")
    wid = lax.axis_index("c") * NS + sid
    base = wid * EW
    pltpu.sync_copy(qtab_hbm, qtab_v)

    pltpu.sync_copy(t1_hbm.at[pl.ds(sid * ROWS_PER_TILE, ROWS_PER_TILE)],
                    ts1.at[pl.ds(sid * ROWS_PER_TILE, ROWS_PER_TILE)])
    pltpu.sync_copy(t2_hbm.at[pl.ds(sid * ROWS_PER_TILE, ROWS_PER_TILE)],
                    ts2.at[pl.ds(sid * ROWS_PER_TILE, ROWS_PER_TILE)])
    plsc.subcore_barrier()

    def extract(j, b, force_load=False):
        cg = wid * NCHUNK + j
        blk = cg // CPB
        cpp = cg % CPB

        def _load():
            pltpu.sync_copy(srcf_hbm.at[pl.ds(blk * EB, EB)], blk1)
            pltpu.sync_copy(tgtf_hbm.at[pl.ds(blk * EB, EB)], blk2)

        if force_load:
            _load()
        else:
            pl.when(cpp == 0)(_load)

        for g in range(CH // 16):
            qv = qtab_v[pl.ds(cpp * CH + 16 * g, 16)]
            idx1[b][pl.ds(16 * g, 16)] = plsc.load_gather(blk1, [qv])
            idx2[b][pl.ds(16 * g, 16)] = plsc.load_gather(blk2, [qv])

    def g_cp(j, b):
        return (pltpu.make_async_copy(ts1.at[idx1[b]], buf1[b], gs1[b]),
                pltpu.make_async_copy(ts2.at[idx2[b]], buf2[b], gs2[b]))

    def w_cp(j, b):
        eb = base + j * CH
        return (pltpu.make_async_copy(buf1[b], g1_hbm.at[pl.ds(eb, CH)], ws1[b]),
                pltpu.make_async_copy(buf2[b], g2_hbm.at[pl.ds(eb, CH)], ws2[b]))

    for b in range(NBUF - 1):
        extract(b, b, force_load=(b == 0))
        for cp in g_cp(b, b):
            cp.start()

    @pl.loop(0, NCHUNK + NBUF - 1 - ((NCHUNK - 1) % NBUF), step=NBUF)
    def _chunks(j):
        for b in range(NBUF):
            jj = j + b
            nxt = jj + NBUF - 1

            @pl.when(jj < NCHUNK)
            def _():
                for cp in g_cp(jj, b):
                    cp.wait()
                for cp in w_cp(jj, b):
                    cp.start()

            @pl.when(nxt < NCHUNK)
            def _():
                bn = (b + NBUF - 1) % NBUF

                @pl.when(nxt >= NBUF)
                def _():
                    for cp in w_cp(nxt - NBUF, bn):
                        cp.wait()

                extract(nxt, bn)
                for cp in g_cp(nxt, bn):
                    cp.start()

    for b in range(NBUF):
        jlast = NCHUNK - NBUF + ((b - NCHUNK) % NBUF)
        for cp in w_cp(jlast, b):
            cp.wait()


def _mlp_body(g1_ref, g2_ref, gx_ref, w_ref, w1a_ref, w1b_ref, b1_ref,
              w2_ref, b2_ref, out_ref):
    cp1 = g1_ref[...]
    cp2 = g2_ref[...]
    g1 = jnp.concatenate([cp1[:, k * CW:(k + 1) * CW] for k in range(C // CW)],
                         axis=0)
    g2 = jnp.concatenate([cp2[:, k * CW:(k + 1) * CW] for k in range(C // CW)],
                         axis=0)
    h = jnp.dot(g1, w1a_ref[...], preferred_element_type=jnp.float32)
    h = h + jnp.dot(g2, w1b_ref[...], preferred_element_type=jnp.float32)
    h = h + b1_ref[...][None, :]
    h = 0.5 * h * (1.0 + lax.erf(h * 0.7071067811865476))
    ker = jnp.dot(h, w2_ref[...], preferred_element_type=jnp.float32)
    ker = ker + b2_ref[...][None, :]
    w = jnp.transpose(w_ref[0], (1, 0))
    out_ref[...] = ker * (w * gx_ref[...])


def _mlp_stage(g1, g2, gx, w, w1a, w1b, b1, W2, b2):
    return pl.pallas_call(
        _mlp_body,
        grid=(GRID,),
        in_specs=[
            pl.BlockSpec((EB * CW // C, C), lambda i: (i, 0)),
            pl.BlockSpec((EB * CW // C, C), lambda i: (i, 0)),
            pl.BlockSpec((EB, C), lambda i: (i, 0)),
            pl.BlockSpec((1, 1, EB), lambda i: (i, 0, 0)),
            pl.BlockSpec((CW, FC), lambda i: (0, 0)),
            pl.BlockSpec((CW, FC), lambda i: (0, 0)),
            pl.BlockSpec((FC,), lambda i: (0,)),
            pl.BlockSpec((FC, C), lambda i: (0, 0)),
            pl.BlockSpec((C,), lambda i: (0,)),
        ],
        out_specs=pl.BlockSpec((EB, C), lambda i: (i, 0)),
        out_shape=jax.ShapeDtypeStruct((E, C), jnp.float32),
        compiler_params=pltpu.CompilerParams(
            dimension_semantics=("arbitrary",),
        ),
    )(g1, g2, gx, w, w1a, w1b, b1, W2, b2)


NBUF_S = 3

@functools.partial(
    pl.kernel,
    out_type=jax.ShapeDtypeStruct((NC, NP, C), jnp.float32),
    mesh=_mesh,
    scratch_types=(
        (pltpu.VMEM((NCHUNK, CH), jnp.int32),
         pltpu.VMEM((16, C), jnp.float32),
         pltpu.VMEM_SHARED((NP, C), jnp.float32),
         pltpu.SemaphoreType.DMA)
        + (pltpu.VMEM((CH, C), jnp.float32),) * NBUF_S
        + (pltpu.SemaphoreType.DMA,) * (2 * NBUF_S)
    ),
)
def _scatter_stage(feat_hbm, tgt_hbm, out_hbm, tgt_v, zbuf, acc, zsem, *bs):
    fbuf, rsems, ssems = bs[:NBUF_S], bs[NBUF_S:2 * NBUF_S], bs[2 * NBUF_S:]
    cid = lax.axis_index("c")
    sid = lax.axis_index("s")
    wid = cid * NS + sid

    for r in range(16):
        for cc in range(C // 16):
            zbuf[r, pl.ds(cc * 16, 16)] = jnp.zeros((16,), jnp.float32)
    row0 = sid * ROWS_PER_TILE

    @pl.loop(0, ROWS_PER_TILE // 16)
    def _zero(j):
        pltpu.async_copy(zbuf, acc.at[pl.ds(row0 + j * 16, 16)], zsem)

    @pl.loop(0, ROWS_PER_TILE // 16)
    def _zdrain(j):
        pltpu.make_async_copy(zbuf, acc.at[pl.ds(row0 + j * 16, 16)],
                              zsem).wait()

    plsc.subcore_barrier()

    pltpu.sync_copy(tgt_hbm.at[wid], tgt_v)
    base = wid * EW

    def r_cp(j, b):
        return pltpu.make_async_copy(
            feat_hbm.at[pl.ds(base + j * CH, CH)], fbuf[b], rsems[b])

    def s_cp(j, b):
        return pltpu.make_async_copy(fbuf[b], acc.at[tgt_v.at[j]], ssems[b])

    for b in range(NBUF_S - 1):
        r_cp(b, b).start()

    @pl.loop(0, NCHUNK + NBUF_S - 1 - ((NCHUNK - 1) % NBUF_S), step=NBUF_S)
    def _chunks(j):
        for b in range(NBUF_S):
            jj = j + b
            nxt = jj + NBUF_S - 1

            @pl.when(jj < NCHUNK)
            def _():
                r_cp(jj, b).wait()
                pltpu.async_copy(fbuf[b], acc.at[tgt_v.at[jj]], ssems[b],
                                 add=True)

            @pl.when(nxt < NCHUNK)
            def _():
                bn = (b + NBUF_S - 1) % NBUF_S

                @pl.when(nxt >= NBUF_S)
                def _():
                    s_cp(nxt - NBUF_S, bn).wait()

                r_cp(nxt, bn).start()

    for b in range(NBUF_S):
        jlast = NCHUNK - NBUF_S + ((b - NCHUNK) % NBUF_S)
        s_cp(jlast, b).wait()

    plsc.subcore_barrier()
    pltpu.sync_copy(acc.at[pl.ds(row0, ROWS_PER_TILE)],
                    out_hbm.at[cid, pl.ds(row0, ROWS_PER_TILE)])


def kernel(nodes_x, nodes_y, x, directed_edges, weights, W1, b1, W2, b2):
    xT = jnp.transpose(x[0], (1, 0))
    pad = jnp.zeros((N, CW - ND), jnp.float32)
    padr = jnp.zeros((NP - N, CW), jnp.float32)
    t1 = jnp.concatenate(
        [jnp.concatenate([nodes_x[0], pad], axis=1), padr], axis=0)
    t2 = jnp.concatenate(
        [jnp.concatenate([nodes_y[0], pad], axis=1), padr], axis=0)

    src = directed_edges[0, :, 0]
    tgt = directed_edges[0, :, 1]
    srcW = src.reshape(NW, NCHUNK, CH)
    tgtW = tgt.reshape(NW, NCHUNK, CH)
    tq, cq = np.meshgrid(np.arange(CH), np.arange(CPB))
    qtab = jnp.asarray(
        (RPB * (tq % 8) + (CH // 8) * cq + tq // 8).reshape(-1).astype(np.int32))

    wpad = jnp.zeros((CW - ND, FC), jnp.float32)
    w1a = jnp.concatenate([W1[:ND], wpad], axis=0)
    w1b = jnp.concatenate([W1[ND:], wpad], axis=0)

    xTp = jnp.concatenate([xT, jnp.zeros((NP - N, C), jnp.float32)], axis=0)
    gx = _gather_x(xTp, srcW)
    g1, g2 = _gather_coords(t1, t2, src, tgt, qtab)
    g1 = g1.reshape(E * CW // C, C)
    g2 = g2.reshape(E * CW // C, C)
    feat = _mlp_stage(g1, g2, gx, weights[0].reshape(GRID, 1, EB), w1a, w1b,
                      b1, W2, b2)
    partial = _scatter_stage(feat, tgtW)
    out = partial[0, :N] + partial[1, :N]
    return jnp.transpose(out, (1, 0))[None]


# speedup vs baseline: 36.5721x; 1.0784x over previous
import functools

import jax
import jax.numpy as jnp
import numpy as np
from jax import lax
from jax.experimental import pallas as pl
from jax.experimental.pallas import tpu as pltpu
from jax.experimental.pallas import tpu_sc as plsc

N = 10000
E = 320000
C = 128
ND = 3
FC = 128

NC, NS = 2, 16
NW = NC * NS
CH = 80

NP = 10240
ROWS_PER_TILE = NP // NS

CW = 16

EB = 6400

SEGS = (128000, 192000)

RPB = EB // 8
CPB = EB // CH

NBUF = 6
NBX = 4
NBUF_S = 3

_mesh = plsc.VectorSubcoreMesh(core_axis_name="c", subcore_axis_name="s")


def _mk_gather_x(ne):
    ew = ne // NW
    nchunk = ew // CH

    @functools.partial(
        pl.kernel,
        out_type=jax.ShapeDtypeStruct((ne, C), jnp.float32),
        mesh=_mesh,
        scratch_types=(
            (pltpu.VMEM_SHARED((NP, C), jnp.float32),)
            + (pltpu.VMEM((CH,), jnp.int32),) * NBX
            + (pltpu.VMEM((CH, C), jnp.float32),) * NBX
            + (pltpu.SemaphoreType.DMA,) * (3 * NBX)
        ),
    )
    def _gather_x(xt_hbm, src_hbm, gx_hbm, xs, *bs):
        idxs, bufs = bs[:NBX], bs[NBX:2 * NBX]
        isems = bs[2 * NBX:3 * NBX]
        gsems = bs[3 * NBX:4 * NBX]
        wsems = bs[4 * NBX:5 * NBX]
        sid = lax.axis_index("s")
        wid = lax.axis_index("c") * NS + sid
        base = wid * ew

        r0 = sid * ROWS_PER_TILE
        pltpu.sync_copy(xt_hbm.at[pl.ds(r0, ROWS_PER_TILE)],
                        xs.at[pl.ds(r0, ROWS_PER_TILE)])
        plsc.subcore_barrier()

        def i_cp(j, b):
            return pltpu.make_async_copy(src_hbm.at[wid, j], idxs[b], isems[b])

        def g_cp(j, b):
            return pltpu.make_async_copy(xs.at[idxs[b]], bufs[b], gsems[b])

        def w_cp(j, b):
            return pltpu.make_async_copy(
                bufs[b], gx_hbm.at[pl.ds(base + j * CH, CH)], wsems[b])

        for b in range(NBX):
            i_cp(b, b).start()
        for b in range(NBX - 1):
            i_cp(b, b).wait()
            g_cp(b, b).start()

        @pl.loop(0, nchunk + NBX - 1 - ((nchunk - 1) % NBX), step=NBX)
        def _chunks(j):
            for b in range(NBX):
                jj = j + b
                nxt = jj + NBX - 1

                @pl.when(jj < nchunk)
                def _():
                    g_cp(jj, b).wait()

                    @pl.when(jj + NBX < nchunk)
                    def _():
                        i_cp(jj + NBX, b).start()

                    w_cp(jj, b).start()

                @pl.when(nxt < nchunk)
                def _():
                    bn = (b + NBX - 1) % NBX

                    @pl.when(nxt >= NBX)
                    def _():
                        w_cp(nxt - NBX, bn).wait()

                    i_cp(nxt, bn).wait()
                    g_cp(nxt, bn).start()

        for b in range(NBX):
            jlast = nchunk - NBX + ((b - nchunk) % NBX)
            w_cp(jlast, b).wait()

    return _gather_x


def _mk_gather_coords(ne):
    ew = ne // NW
    nchunk = ew // CH

    @functools.partial(
        pl.kernel,
        out_type=(
            jax.ShapeDtypeStruct((ne, CW), jnp.float32),
            jax.ShapeDtypeStruct((ne, CW), jnp.float32),
        ),
        mesh=_mesh,
        scratch_types=(
            (pltpu.VMEM_SHARED((NP, CW), jnp.float32),) * 2
            + (pltpu.VMEM((EB,), jnp.int32),) * 2
            + (pltpu.VMEM((CPB * CH,), jnp.int32),)
            + (pltpu.VMEM((CH,), jnp.int32),) * (2 * NBUF)
            + (pltpu.VMEM((CH, CW), jnp.float32),) * (2 * NBUF)
            + (pltpu.SemaphoreType.DMA,) * (4 * NBUF)
        ),
        compiler_params=pltpu.CompilerParams(use_tc_tiling_on_sc=False,
                                             needs_layout_passes=False),
    )
    def _gather_coords(t1_hbm, t2_hbm, srcf_hbm, tgtf_hbm, qtab_hbm,
                       g1_hbm, g2_hbm, ts1, ts2, blk1, blk2, qtab_v, *bs):
        idx1, idx2 = bs[:NBUF], bs[NBUF:2 * NBUF]
        buf1, buf2 = bs[2 * NBUF:3 * NBUF], bs[3 * NBUF:4 * NBUF]
        gs1 = bs[4 * NBUF:5 * NBUF]
        gs2 = bs[5 * NBUF:6 * NBUF]
        ws1 = bs[6 * NBUF:7 * NBUF]
        ws2 = bs[7 * NBUF:8 * NBUF]
        sid = lax.axis_index("s")
        wid = lax.axis_index("c") * NS + sid
        base = wid * ew
        pltpu.sync_copy(qtab_hbm, qtab_v)

        pltpu.sync_copy(t1_hbm.at[pl.ds(sid * ROWS_PER_TILE, ROWS_PER_TILE)],
                        ts1.at[pl.ds(sid * ROWS_PER_TILE, ROWS_PER_TILE)])
        pltpu.sync_copy(t2_hbm.at[pl.ds(sid * ROWS_PER_TILE, ROWS_PER_TILE)],
                        ts2.at[pl.ds(sid * ROWS_PER_TILE, ROWS_PER_TILE)])
        plsc.subcore_barrier()

        def extract(j, b, force_load=False):
            cg = wid * nchunk + j
            blk = cg // CPB
            cpp = cg % CPB

            def _load():
                pltpu.sync_copy(srcf_hbm.at[pl.ds(blk * EB, EB)], blk1)
                pltpu.sync_copy(tgtf_hbm.at[pl.ds(blk * EB, EB)], blk2)

            if force_load:
                _load()
            else:
                pl.when(cpp == 0)(_load)

            for g in range(CH // 16):
                qv = qtab_v[pl.ds(cpp * CH + 16 * g, 16)]
                idx1[b][pl.ds(16 * g, 16)] = plsc.load_gather(blk1, [qv])
                idx2[b][pl.ds(16 * g, 16)] = plsc.load_gather(blk2, [qv])

        def g_cp(j, b):
            return (pltpu.make_async_copy(ts1.at[idx1[b]], buf1[b], gs1[b]),
                    pltpu.make_async_copy(ts2.at[idx2[b]], buf2[b], gs2[b]))

        def w_cp(j, b):
            eb = base + j * CH
            return (
                pltpu.make_async_copy(buf1[b], g1_hbm.at[pl.ds(eb, CH)], ws1[b]),
                pltpu.make_async_copy(buf2[b], g2_hbm.at[pl.ds(eb, CH)], ws2[b]))

        for b in range(NBUF - 1):
            extract(b, b, force_load=(b == 0))
            for cp in g_cp(b, b):
                cp.start()

        @pl.loop(0, nchunk + NBUF - 1 - ((nchunk - 1) % NBUF), step=NBUF)
        def _chunks(j):
            for b in range(NBUF):
                jj = j + b
                nxt = jj + NBUF - 1

                @pl.when(jj < nchunk)
                def _():
                    for cp in g_cp(jj, b):
                        cp.wait()
                    for cp in w_cp(jj, b):
                        cp.start()

                @pl.when(nxt < nchunk)
                def _():
                    bn = (b + NBUF - 1) % NBUF

                    @pl.when(nxt >= NBUF)
                    def _():
                        for cp in w_cp(nxt - NBUF, bn):
                            cp.wait()

                    extract(nxt, bn)
                    for cp in g_cp(nxt, bn):
                        cp.start()

        for b in range(NBUF):
            jlast = nchunk - NBUF + ((b - nchunk) % NBUF)
            for cp in w_cp(jlast, b):
                cp.wait()

    return _gather_coords


def _mlp_body(g1_ref, g2_ref, gx_ref, w_ref, w1a_ref, w1b_ref, b1_ref,
              w2_ref, b2_ref, out_ref):
    cp1 = g1_ref[...]
    cp2 = g2_ref[...]
    g1 = jnp.concatenate([cp1[:, k * CW:(k + 1) * CW] for k in range(C // CW)],
                         axis=0)
    g2 = jnp.concatenate([cp2[:, k * CW:(k + 1) * CW] for k in range(C // CW)],
                         axis=0)
    h = jnp.dot(g1, w1a_ref[...], preferred_element_type=jnp.float32)
    h = h + jnp.dot(g2, w1b_ref[...], preferred_element_type=jnp.float32)
    h = h + b1_ref[...][None, :]
    h = 0.5 * h * (1.0 + lax.erf(h * 0.7071067811865476))
    ker = jnp.dot(h, w2_ref[...], preferred_element_type=jnp.float32)
    ker = ker + b2_ref[...][None, :]
    w = jnp.transpose(w_ref[0], (1, 0))
    out_ref[...] = ker * (w * gx_ref[...])


def _mlp_stage(g1, g2, gx, w, w1a, w1b, b1, W2, b2):
    ne = gx.shape[0]
    return pl.pallas_call(
        _mlp_body,
        grid=(ne // EB,),
        in_specs=[
            pl.BlockSpec((EB * CW // C, C), lambda i: (i, 0)),
            pl.BlockSpec((EB * CW // C, C), lambda i: (i, 0)),
            pl.BlockSpec((EB, C), lambda i: (i, 0)),
            pl.BlockSpec((1, 1, EB), lambda i: (i, 0, 0)),
            pl.BlockSpec((CW, FC), lambda i: (0, 0)),
            pl.BlockSpec((CW, FC), lambda i: (0, 0)),
            pl.BlockSpec((FC,), lambda i: (0,)),
            pl.BlockSpec((FC, C), lambda i: (0, 0)),
            pl.BlockSpec((C,), lambda i: (0,)),
        ],
        out_specs=pl.BlockSpec((EB, C), lambda i: (i, 0)),
        out_shape=jax.ShapeDtypeStruct((ne, C), jnp.float32),
        compiler_params=pltpu.CompilerParams(
            dimension_semantics=("arbitrary",),
        ),
    )(g1, g2, gx, w, w1a, w1b, b1, W2, b2)


def _mk_scatter(ne):
    ew = ne // NW
    nchunk = ew // CH

    @functools.partial(
        pl.kernel,
        out_type=jax.ShapeDtypeStruct((NC, NP, C), jnp.float32),
        mesh=_mesh,
        scratch_types=(
            (pltpu.VMEM((nchunk, CH), jnp.int32),
             pltpu.VMEM((16, C), jnp.float32),
             pltpu.VMEM_SHARED((NP, C), jnp.float32),
             pltpu.SemaphoreType.DMA)
            + (pltpu.VMEM((CH, C), jnp.float32),) * NBUF_S
            + (pltpu.SemaphoreType.DMA,) * (2 * NBUF_S)
        ),
    )
    def _scatter_stage(feat_hbm, tgt_hbm, out_hbm, tgt_v, zbuf, acc, zsem, *bs):
        fbuf = bs[:NBUF_S]
        rsems, ssems = bs[NBUF_S:2 * NBUF_S], bs[2 * NBUF_S:]
        cid = lax.axis_index("c")
        sid = lax.axis_index("s")
        wid = cid * NS + sid

        for r in range(16):
            for cc in range(C // 16):
                zbuf[r, pl.ds(cc * 16, 16)] = jnp.zeros((16,), jnp.float32)
        row0 = sid * ROWS_PER_TILE

        @pl.loop(0, ROWS_PER_TILE // 16)
        def _zero(j):
            pltpu.async_copy(zbuf, acc.at[pl.ds(row0 + j * 16, 16)], zsem)

        @pl.loop(0, ROWS_PER_TILE // 16)
        def _zdrain(j):
            pltpu.make_async_copy(zbuf, acc.at[pl.ds(row0 + j * 16, 16)],
                                  zsem).wait()

        plsc.subcore_barrier()

        pltpu.sync_copy(tgt_hbm.at[wid], tgt_v)
        base = wid * ew

        def r_cp(j, b):
            return pltpu.make_async_copy(
                feat_hbm.at[pl.ds(base + j * CH, CH)], fbuf[b], rsems[b])

        def s_cp(j, b):
            return pltpu.make_async_copy(fbuf[b], acc.at[tgt_v.at[j]],
                                         ssems[b])

        for b in range(NBUF_S - 1):
            r_cp(b, b).start()

        @pl.loop(0, nchunk + NBUF_S - 1 - ((nchunk - 1) % NBUF_S), step=NBUF_S)
        def _chunks(j):
            for b in range(NBUF_S):
                jj = j + b
                nxt = jj + NBUF_S - 1

                @pl.when(jj < nchunk)
                def _():
                    r_cp(jj, b).wait()
                    pltpu.async_copy(fbuf[b], acc.at[tgt_v.at[jj]], ssems[b],
                                     add=True)

                @pl.when(nxt < nchunk)
                def _():
                    bn = (b + NBUF_S - 1) % NBUF_S

                    @pl.when(nxt >= NBUF_S)
                    def _():
                        s_cp(nxt - NBUF_S, bn).wait()

                    r_cp(nxt, bn).start()

        for b in range(NBUF_S):
            jlast = nchunk - NBUF_S + ((b - nchunk) % NBUF_S)
            s_cp(jlast, b).wait()

        plsc.subcore_barrier()
        pltpu.sync_copy(acc.at[pl.ds(row0, ROWS_PER_TILE)],
                        out_hbm.at[cid, pl.ds(row0, ROWS_PER_TILE)])

    return _scatter_stage


_GATHER_X = {ne: _mk_gather_x(ne) for ne in set(SEGS)}
_GATHER_C = {ne: _mk_gather_coords(ne) for ne in set(SEGS)}
_SCATTER = {ne: _mk_scatter(ne) for ne in set(SEGS)}


def kernel(nodes_x, nodes_y, x, directed_edges, weights, W1, b1, W2, b2):
    xT = jnp.transpose(x[0], (1, 0))
    pad = jnp.zeros((N, CW - ND), jnp.float32)
    padr = jnp.zeros((NP - N, CW), jnp.float32)
    t1 = jnp.concatenate(
        [jnp.concatenate([nodes_x[0], pad], axis=1), padr], axis=0)
    t2 = jnp.concatenate(
        [jnp.concatenate([nodes_y[0], pad], axis=1), padr], axis=0)
    xTp = jnp.concatenate([xT, jnp.zeros((NP - N, C), jnp.float32)], axis=0)

    src = directed_edges[0, :, 0]
    tgt = directed_edges[0, :, 1]
    tq, cq = np.meshgrid(np.arange(CH), np.arange(CPB))
    qtab = jnp.asarray(
        (RPB * (tq % 8) + (CH // 8) * cq + tq // 8).reshape(-1).astype(np.int32))

    wpad = jnp.zeros((CW - ND, FC), jnp.float32)
    w1a = jnp.concatenate([W1[:ND], wpad], axis=0)
    w1b = jnp.concatenate([W1[ND:], wpad], axis=0)

    partials = []
    off = 0
    for ne in SEGS:
        src_s = lax.dynamic_slice_in_dim(src, off, ne)
        tgt_s = lax.dynamic_slice_in_dim(tgt, off, ne)
        w_s = lax.dynamic_slice_in_dim(weights[0], off, ne)
        srcW = src_s.reshape(NW, ne // NW // CH, CH)
        tgtW = tgt_s.reshape(NW, ne // NW // CH, CH)

        gx = _GATHER_X[ne](xTp, srcW)
        g1, g2 = _GATHER_C[ne](t1, t2, src_s, tgt_s, qtab)
        g1 = g1.reshape(ne * CW // C, C)
        g2 = g2.reshape(ne * CW // C, C)
        feat = _mlp_stage(g1, g2, gx, w_s.reshape(ne // EB, 1, EB),
                          w1a, w1b, b1, W2, b2)
        partials.append(_SCATTER[ne](feat, tgtW))
        off += ne

    out = sum(p[0, :N] + p[1, :N] for p in partials)
    return jnp.transpose(out, (1, 0))[None]
